# baseline (speedup 1.0000x reference)
import functools

import jax
import jax.numpy as jnp
from jax import lax
from jax.experimental import pallas as pl
from jax.experimental.pallas import tpu as pltpu
from jax.experimental.pallas import tpu_sc as plsc

_EPS = 1e-5
_NC = 2
_NS = 16
_NW = _NC * _NS
_CH = 80

_F32 = jnp.float32
_BF16 = jnp.bfloat16


def _bfr(x):
    return x.astype(_BF16).astype(_F32)


def _node_prologue_body(pv_ref, Wp_ref, bp_ref, h_ref):
    h = jnp.dot(pv_ref[...].astype(_BF16), Wp_ref[...].astype(_BF16),
                preferred_element_type=_F32)
    h_ref[...] = h + bp_ref[0]


def _stats_accum(i, y, ps_ref, pq_ref):
    yr = y.reshape(-1, 8, 128)
    ps = jnp.sum(yr, axis=0)
    pq = jnp.sum(yr * yr, axis=0)

    @pl.when(i == 0)
    def _init():
        ps_ref[...] = ps
        pq_ref[...] = pq

    @pl.when(i != 0)
    def _acc():
        ps_ref[...] = ps_ref[...] + ps
        pq_ref[...] = pq_ref[...] + pq


def _edge_stats_body(x_ref, W_ref, ps_ref, pq_ref):
    y = jnp.dot(x_ref[...].astype(_BF16), W_ref[...],
                preferred_element_type=_F32)
    _stats_accum(pl.program_id(0), y, ps_ref, pq_ref)


def _edge_tmm_body(x_ref, p_ref, W_ref, t_ref, cs_ref, G_ref):
    i = pl.program_id(0)
    xv = x_ref[...]
    if xv.dtype != _BF16:
        xv = xv.astype(_BF16)
    x = jnp.dot(xv, W_ref[...], preferred_element_type=_F32) + p_ref[2]
    t16 = jnp.tanh(x * p_ref[0] + p_ref[1]).astype(_BF16)
    t_ref[...] = t16
    tf = t16.astype(_F32)
    yr = tf.reshape(-1, 8, 128)
    cs = jnp.sum(yr, axis=0)
    G = lax.dot_general(t16, t16, (((0,), (0,)), ((), ())),
                        preferred_element_type=_F32)

    @pl.when(i == 0)
    def _init():
        cs_ref[...] = cs
        G_ref[...] = G

    @pl.when(i != 0)
    def _acc():
        cs_ref[...] = cs_ref[...] + cs
        G_ref[...] = G_ref[...] + G


def _edge_last_body(x_ref, p_ref, W_ref, t_ref):
    x = jnp.dot(x_ref[...], W_ref[...],
                preferred_element_type=_F32) + p_ref[2]
    t_ref[...] = _bfr(jnp.tanh(x * p_ref[0] + p_ref[1]))


def _finale_body(S_ref, C_ref, Wf0_ref, Wf1_ref, mbf_ref, uW0_ref, uWa_ref,
                 uWb_ref, Wpred_ref, P_ref, out_ref):
    S = S_ref[...]
    cnt = C_ref[:, 0:1]

    hi = functools.partial(jnp.dot, preferred_element_type=_F32,
                           precision=lax.Precision.HIGHEST)
    agg_a = hi(S, Wf0_ref[...]) + cnt * mbf_ref[0]
    agg_b = (hi(S, Wf1_ref[...]) + cnt * mbf_ref[1]) / jnp.maximum(cnt, 1.0)
    x = agg_a + agg_b

    def bn(v, gm, bt):
        m = jnp.mean(v, axis=0)
        q = jnp.mean(v * v, axis=0)
        s = gm * lax.rsqrt(q - m * m + _EPS)
        return (v - m) * s + bt

    def mm(a, W_ref):
        return jnp.dot(a.astype(_BF16), W_ref[...],
                       preferred_element_type=_F32)

    x = mm(x, uW0_ref) + P_ref[0]
    x = mm(jnp.tanh(bn(x, P_ref[3], P_ref[4])), uWa_ref) + P_ref[1]
    x = mm(jnp.tanh(bn(x, P_ref[3], P_ref[4])), uWa_ref) + P_ref[1]
    x = mm(jnp.tanh(bn(x, P_ref[5], P_ref[6])), uWb_ref) + P_ref[2]
    x = jnp.tanh(bn(x, P_ref[7], P_ref[8]))
    out_ref[...] = mm(x, Wpred_ref) + P_ref[9]


def _edge_stats_pass(x, W16, block_e):
    E = x.shape[0]
    return pl.pallas_call(
        _edge_stats_body,
        grid=(E // block_e,),
        in_specs=[
            pl.BlockSpec((block_e, 128), lambda i: (i, 0)),
            pl.BlockSpec((128, 128), lambda i: (0, 0)),
        ],
        out_specs=[
            pl.BlockSpec((8, 128), lambda i: (0, 0)),
            pl.BlockSpec((8, 128), lambda i: (0, 0)),
        ],
        out_shape=[
            jax.ShapeDtypeStruct((8, 128), _F32),
            jax.ShapeDtypeStruct((8, 128), _F32),
        ],
        compiler_params=pltpu.CompilerParams(
            dimension_semantics=("arbitrary",)),
    )(x, W16)


def _edge_tmm_pass(x, params, W16, block_e):
    E = x.shape[0]
    return pl.pallas_call(
        _edge_tmm_body,
        grid=(E // block_e,),
        in_specs=[
            pl.BlockSpec((block_e, 128), lambda i: (i, 0)),
            pl.BlockSpec((8, 128), lambda i: (0, 0)),
            pl.BlockSpec((128, 128), lambda i: (0, 0)),
        ],
        out_specs=[
            pl.BlockSpec((block_e, 128), lambda i: (i, 0)),
            pl.BlockSpec((8, 128), lambda i: (0, 0)),
            pl.BlockSpec((128, 128), lambda i: (0, 0)),
        ],
        out_shape=[
            jax.ShapeDtypeStruct((E, 128), _BF16),
            jax.ShapeDtypeStruct((8, 128), _F32),
            jax.ShapeDtypeStruct((128, 128), _F32),
        ],
        compiler_params=pltpu.CompilerParams(
            dimension_semantics=("arbitrary",)),
    )(x, params, W16)


def _edge_last_pass(x, params, W16, block_e):
    E = x.shape[0]
    return pl.pallas_call(
        _edge_last_body,
        grid=(E // block_e,),
        in_specs=[
            pl.BlockSpec((block_e, 128), lambda i: (i, 0)),
            pl.BlockSpec((8, 128), lambda i: (0, 0)),
            pl.BlockSpec((128, 128), lambda i: (0, 0)),
        ],
        out_specs=pl.BlockSpec((block_e, 128), lambda i: (i, 0)),
        out_shape=jax.ShapeDtypeStruct((E, 128), _F32),
        compiler_params=pltpu.CompilerParams(
            dimension_semantics=("arbitrary",)),
    )(x, params, W16)


def _sc_gather_diff(h, src2d, dst2d):
    NCHT, CH = src2d.shape
    E = NCHT * CH
    EW = E // _NW
    NCH = EW // CH
    mesh = plsc.VectorSubcoreMesh(core_axis_name="c", subcore_axis_name="s")

    @functools.partial(
        pl.kernel,
        out_type=jax.ShapeDtypeStruct((E, 128), _BF16),
        mesh=mesh,
        scratch_types=[
            pltpu.VMEM((NCH, CH), jnp.int32),
            pltpu.VMEM((NCH, CH), jnp.int32),
            pltpu.VMEM((CH, 128), _F32),
            pltpu.VMEM((CH, 128), _F32),
            pltpu.VMEM((CH, 128), _F32),
            pltpu.VMEM((CH, 128), _F32),
            pltpu.VMEM((CH, 128), _F32),
            pltpu.VMEM((CH, 128), _F32),
            pltpu.VMEM((CH, 128), _BF16),
            pltpu.VMEM((CH, 128), _BF16),
            pltpu.VMEM((CH, 128), _BF16),
            pltpu.SemaphoreType.DMA,
            pltpu.SemaphoreType.DMA,
            pltpu.SemaphoreType.DMA,
            pltpu.SemaphoreType.DMA,
            pltpu.SemaphoreType.DMA,
            pltpu.SemaphoreType.DMA,
        ],
        compiler_params=pltpu.CompilerParams(use_tc_tiling_on_sc=False,
                                             needs_layout_passes=False),
    )
    def k(h_hbm, src_hbm, dst_hbm, d_hbm, isrc, idst, gs0, gs1, gs2,
          gd0, gd1, gd2, db0, db1, db2, semg0, semg1, semg2,
          semw0, semw1, semw2):
        gs = (gs0, gs1, gs2)
        gd = (gd0, gd1, gd2)
        db = (db0, db1, db2)
        semg = (semg0, semg1, semg2)
        semw = (semw0, semw1, semw2)
        wid = lax.axis_index("s") * _NC + lax.axis_index("c")
        base = wid * EW
        row0 = wid * NCH

        pltpu.sync_copy(src_hbm.at[pl.ds(row0, NCH)], isrc)
        pltpu.sync_copy(dst_hbm.at[pl.ds(row0, NCH)], idst)

        def start_gathers(ci, s):
            pltpu.async_copy(h_hbm.at[isrc.at[ci]], gs[s], semg[s])
            pltpu.async_copy(h_hbm.at[idst.at[ci]], gd[s], semg[s])

        def wait_gathers(s):
            pltpu.make_async_copy(h_hbm.at[isrc.at[0]], gs[s], semg[s]).wait()
            pltpu.make_async_copy(h_hbm.at[idst.at[0]], gd[s], semg[s]).wait()

        def wait_write(s, ci):
            off = pl.multiple_of(base + ci * CH, 8)
            pltpu.make_async_copy(db[s], d_hbm.at[pl.ds(off, CH)],
                                  semw[s]).wait()

        start_gathers(0, 0)
        start_gathers(1, 1)

        def tri_body(t, _):
            for b in range(3):
                ci = 3 * t + b
                p = (b + 2) % 3

                @pl.when(ci < NCH)
                def _work():
                    wait_gathers(b)

                    @pl.when(ci >= 3)
                    def _ww():
                        wait_write(b, ci - 3)

                    def row_body(r, _):
                        for q in range(4):
                            sa = pl.ds(q * 32, 16)
                            sb = pl.ds(q * 32 + 16, 16)
                            da = gd[b][r, sa] - gs[b][r, sa]
                            dbv = gd[b][r, sb] - gs[b][r, sb]
                            db[b][r, pl.ds(q * 32, 32)] = plsc.pack(
                                da, dbv, format=plsc.PackFormat.INTERLEAVED)
                        return 0

                    lax.fori_loop(0, CH, row_body, 0)
                    off = pl.multiple_of(base + ci * CH, 8)
                    pltpu.async_copy(db[b], d_hbm.at[pl.ds(off, CH)], semw[b])

                    @pl.when(ci + 2 < NCH)
                    def _pf():
                        start_gathers(ci + 2, p)

            return 0

        lax.fori_loop(0, (NCH + 2) // 3, tri_body, 0)
        for last in range(max(NCH - 3, 0), NCH):
            wait_write(last % 3, last)

    return k(h, src2d, dst2d)


def _sc_scatter_add(t3, dst2d, n_nodes):
    NCHT, CH = dst2d.shape
    E = NCHT * CH
    EW = E // _NS
    NCH = EW // CH
    H = n_nodes // _NC
    HP = H + 128
    RPT = H // _NS // 8 * 8
    RLAST = H - (_NS - 1) * RPT
    CPT = RPT // 8
    CLAST = RLAST // 8
    mesh = plsc.VectorSubcoreMesh(core_axis_name="c", subcore_axis_name="s")

    @functools.partial(
        pl.kernel,
        out_type=[
            jax.ShapeDtypeStruct((n_nodes, 128), _F32),
            jax.ShapeDtypeStruct((n_nodes // 8, 128), _F32),
        ],
        mesh=mesh,
        scratch_types=[
            pltpu.VMEM((NCH, CH), jnp.int32),
            pltpu.VMEM((CH, 128), _F32),
            pltpu.VMEM((CH, 128), _F32),
            pltpu.VMEM((CH, 128), _F32),
            pltpu.VMEM((CH, 16), _F32),
            pltpu.VMEM((RLAST, 16), _F32),
            pltpu.VMEM((CLAST, 128), _F32),
            pltpu.VMEM_SHARED((HP, 128), _F32),
            pltpu.VMEM_SHARED((HP, 16), _F32),
            pltpu.SemaphoreType.DMA,
            pltpu.SemaphoreType.DMA,
            pltpu.SemaphoreType.DMA,
            pltpu.SemaphoreType.DMA,
            pltpu.SemaphoreType.DMA,
            pltpu.SemaphoreType.DMA,
            pltpu.SemaphoreType.DMA,
            pltpu.SemaphoreType.DMA,
            pltpu.SemaphoreType.DMA,
        ],
        compiler_params=pltpu.CompilerParams(use_tc_tiling_on_sc=False),
    )
    def k(t3_hbm, dst_hbm, zS_hbm, zC_hbm, outS, outC,
          idxa, tbuf0, tbuf1, tbuf2, onesv, bC, bC2, S_sh, C_sh,
          semt0, semt1, semt2, sems0, sems1, sems2, semo0, semo1, semo2):
        tbuf = (tbuf0, tbuf1, tbuf2)
        semt = (semt0, semt1, semt2)
        sems = (sems0, sems1, sems2)
        semo = (semo0, semo1, semo2)
        cid = lax.axis_index("c")
        sid = lax.axis_index("s")
        node0 = cid * H
        ro = sid * RPT
        row0 = sid * NCH

        one = jnp.full((16,), 1.0, _F32)

        def ones_body(r, _):
            onesv[r, :] = one
            return 0

        lax.fori_loop(0, CH, ones_body, 0)

        @pl.when(sid == 0)
        def _z0():
            pltpu.sync_copy(zS_hbm, S_sh)
            pltpu.sync_copy(zC_hbm, C_sh)

        pltpu.sync_copy(dst_hbm.at[pl.ds(row0, NCH)], idxa)

        def adj_body(r, _):
            for j in range(CH // 16):
                sl = pl.ds(j * 16, 16)
                local = idxa[r, sl] - node0
                ok = (local >= 0) & (local < H)
                idxa[r, sl] = jnp.where(ok, local, H)
            return 0

        lax.fori_loop(0, NCH, adj_body, 0)
        plsc.subcore_barrier()

        def load_t3(ci, s):
            off = pl.multiple_of(sid * EW + ci * CH, 8)
            pltpu.async_copy(t3_hbm.at[pl.ds(off, CH)], tbuf[s], semt[s])

        def wait_load(s):
            pltpu.make_async_copy(t3_hbm.at[pl.ds(0, CH)], tbuf[s],
                                  semt[s]).wait()

        def wait_scats(s):
            pltpu.make_async_copy(tbuf[s], S_sh.at[pl.ds(0, CH)],
                                  sems[s]).wait()
            pltpu.make_async_copy(onesv, C_sh.at[pl.ds(0, CH)],
                                  semo[s]).wait()

        load_t3(0, 0)
        load_t3(1, 1)

        def tri_body(t, _):
            for b in range(3):
                ci = 3 * t + b
                p = (b + 2) % 3

                @pl.when(ci < NCH)
                def _work():
                    wait_load(b)
                    pltpu.async_copy(tbuf[b], S_sh.at[idxa.at[ci]], sems[b],
                                     add=True)
                    pltpu.async_copy(onesv, C_sh.at[idxa.at[ci]], semo[b],
                                     add=True)

                    @pl.when(ci + 2 < NCH)
                    def _pf():
                        @pl.when(ci >= 1)
                        def _ws():
                            wait_scats(p)

                        load_t3(ci + 2, p)

            return 0

        lax.fori_loop(0, (NCH + 2) // 3, tri_body, 0)
        for last in range(max(NCH - 3, 0), NCH):
            wait_scats(last % 3)
        plsc.subcore_barrier()

        @pl.when(sid != _NS - 1)
        def _r0():
            pltpu.sync_copy(S_sh.at[pl.ds(ro, RPT)],
                            outS.at[pl.ds(node0 + ro, RPT)])
            pltpu.sync_copy(C_sh.at[pl.ds(ro, RPT)], bC.at[pl.ds(0, RPT)])

        @pl.when(sid == _NS - 1)
        def _r1():
            pltpu.sync_copy(S_sh.at[pl.ds(ro, RLAST)],
                            outS.at[pl.ds(node0 + ro, RLAST)])
            pltpu.sync_copy(C_sh.at[pl.ds(ro, RLAST)], bC)

        def repack_body(r, _):
            bC2[r // 8, pl.ds((r % 8) * 16, 16)] = bC[r, :]
            return 0

        lax.fori_loop(0, RLAST, repack_body, 0)
        co = cid * (H // 8) + sid * CPT

        @pl.when(sid != _NS - 1)
        def _c0():
            pltpu.sync_copy(bC2.at[pl.ds(0, CPT)], outC.at[pl.ds(co, CPT)])

        @pl.when(sid == _NS - 1)
        def _c1():
            pltpu.sync_copy(bC2, outC.at[pl.ds(co, CLAST)])

    zS = jnp.zeros((HP, 128), _F32)
    zC = jnp.zeros((HP, 16), _F32)
    return k(t3, dst2d, zS, zC)


def _affine_from_stats(ps, pq, count, gamma, beta):
    m = jnp.sum(ps, 0) / count
    var = jnp.sum(pq, 0) / count - m * m
    scale = gamma * lax.rsqrt(var + _EPS)
    return scale, beta - m * scale


def _affine_from_gram(cs, G, Wr, badd, count, gamma, beta):
    hi = functools.partial(jnp.dot, precision=lax.Precision.HIGHEST)
    mu = hi(jnp.sum(cs, 0) / count, Wr)
    T = hi(G / count, Wr)
    ex2 = jnp.sum(Wr * T, axis=0)
    var = ex2 - mu * mu
    scale = gamma * lax.rsqrt(var + _EPS)
    return scale, beta - (mu + badd) * scale


def kernel(pos, vel, edge_index, W_in, b_in, mW0, mb0, mWa, mba, mWf, mbf,
           mg_a, mbe_a, mg_b, mbe_b, uW0, ub0, uWa, uba, uWb, ubb,
           ug_a, ube_a, ug_b, ube_b, ug_c, ube_c, W_pred, b_pred):
    N = pos.shape[0]
    E = edge_index.shape[1]
    src = edge_index[0]
    dst = edge_index[1]
    BE = 2560
    fE = jnp.float32(E)

    pv = jnp.zeros((N, 128), _F32)
    pv = pv.at[:, 0:2].set(pos).at[:, 2:4].set(vel)
    Wp = jnp.zeros((128, 128), _F32).at[0:4, :].set(W_in)
    h = pl.pallas_call(
        _node_prologue_body,
        out_shape=jax.ShapeDtypeStruct((N, 128), _F32),
    )(pv, Wp, b_in.reshape(1, 128))

    src2d = src.reshape(E // _CH, _CH)
    dst2d = dst.reshape(E // _CH, _CH)
    d = _sc_gather_diff(h, src2d, dst2d)

    def pack_p(scale, bias, bout):
        p = jnp.zeros((8, 128), _F32)
        return p.at[0].set(scale).at[1].set(bias).at[2].set(bout)

    perm = []
    for q in range(4):
        for j in range(16):
            perm.append(q * 32 + j)
            perm.append(q * 32 + 16 + j)
    mW0_16 = mW0[jnp.array(perm, jnp.int32), :].astype(_BF16)
    mWa_16 = mWa.astype(_BF16)
    zeros128 = jnp.zeros((128,), _F32)

    ps1, pq1 = _edge_stats_pass(d, mW0_16, BE)
    s1, b1 = _affine_from_stats(ps1, pq1, fE, mg_a, mbe_a)

    t1b, cs2, G2 = _edge_tmm_pass(d, pack_p(s1, b1, zeros128), mW0_16, BE)
    s2, b2 = _affine_from_gram(cs2, G2, _bfr(mWa), mba, fE, mg_a, mbe_a)

    t2b, cs3, G3 = _edge_tmm_pass(t1b, pack_p(s2, b2, mba), mWa_16, BE)
    s3, b3 = _affine_from_gram(cs3, G3, _bfr(mWa), mba, fE, mg_b, mbe_b)

    t3 = _edge_last_pass(t2b, pack_p(s3, b3, mba), mWa_16, BE)

    S_seg, C_raw = _sc_scatter_add(t3, dst2d, N)
    cnt = jnp.broadcast_to(C_raw.reshape(N, 16)[:, 0:1], (N, 8))

    P = jnp.zeros((16, 128), _F32)
    for i, v in enumerate([ub0, uba, ubb, ug_a, ube_a, ug_b, ube_b,
                           ug_c, ube_c]):
        P = P.at[i].set(v)
    P = P.at[9, 0:2].set(b_pred)
    Wpred16 = jnp.zeros((128, 128), _F32).at[:, 0:2].set(W_pred).astype(_BF16)
    out = pl.pallas_call(
        _finale_body,
        out_shape=jax.ShapeDtypeStruct((N, 128), _F32),
    )(S_seg, cnt, _bfr(mWf[:, :128]), _bfr(mWf[:, 128:]),
      mbf.reshape(2, 128), uW0.astype(_BF16), uWa.astype(_BF16),
      uWb.astype(_BF16), Wpred16, P)
    return out[:, 0:2]


# speedup vs baseline: 1.3473x; 1.3473x over previous
import functools

import jax
import jax.numpy as jnp
from jax import lax
from jax.experimental import pallas as pl
from jax.experimental.pallas import tpu as pltpu
from jax.experimental.pallas import tpu_sc as plsc

_EPS = 1e-5
_NC = 2
_NS = 16
_NW = _NC * _NS
_CH = 80

_F32 = jnp.float32
_BF16 = jnp.bfloat16


def _bfr(x):
    return x.astype(_BF16).astype(_F32)


def _node_prologue_body(pv_ref, Wp_ref, bp_ref, h_ref):
    h = jnp.dot(pv_ref[...].astype(_BF16), Wp_ref[...].astype(_BF16),
                preferred_element_type=_F32)
    h_ref[...] = h + bp_ref[0]


def _stats_accum(i, y, ps_ref, pq_ref):
    yr = y.reshape(-1, 8, 128)
    ps = jnp.sum(yr, axis=0)
    pq = jnp.sum(yr * yr, axis=0)

    @pl.when(i == 0)
    def _init():
        ps_ref[...] = ps
        pq_ref[...] = pq

    @pl.when(i != 0)
    def _acc():
        ps_ref[...] = ps_ref[...] + ps
        pq_ref[...] = pq_ref[...] + pq


def _edge_stats_body(x_ref, W_ref, ps_ref, pq_ref):
    y = jnp.dot(x_ref[...].astype(_BF16), W_ref[...],
                preferred_element_type=_F32)
    _stats_accum(pl.program_id(0), y, ps_ref, pq_ref)


def _edge_tmm_body(x_ref, p_ref, W_ref, t_ref, cs_ref, G_ref):
    i = pl.program_id(0)
    xv = x_ref[...]
    if xv.dtype != _BF16:
        xv = xv.astype(_BF16)
    x = jnp.dot(xv, W_ref[...], preferred_element_type=_F32) + p_ref[2]
    t16 = jnp.tanh(x * p_ref[0] + p_ref[1]).astype(_BF16)
    t_ref[...] = t16
    tf = t16.astype(_F32)
    yr = tf.reshape(-1, 8, 128)
    cs = jnp.sum(yr, axis=0)
    G = lax.dot_general(t16, t16, (((0,), (0,)), ((), ())),
                        preferred_element_type=_F32)

    @pl.when(i == 0)
    def _init():
        cs_ref[...] = cs
        G_ref[...] = G

    @pl.when(i != 0)
    def _acc():
        cs_ref[...] = cs_ref[...] + cs
        G_ref[...] = G_ref[...] + G


def _edge_last_body(x_ref, p_ref, W_ref, t_ref):
    x = jnp.dot(x_ref[...], W_ref[...],
                preferred_element_type=_F32) + p_ref[2]
    t_ref[...] = _bfr(jnp.tanh(x * p_ref[0] + p_ref[1]))


def _finale_body(S_ref, C_ref, Wf0_ref, Wf1_ref, mbf_ref, uW0_ref, uWa_ref,
                 uWb_ref, Wpred_ref, P_ref, out_ref):
    S = S_ref[...]
    cnt = C_ref[:, 0:1]

    hi = functools.partial(jnp.dot, preferred_element_type=_F32,
                           precision=lax.Precision.HIGHEST)
    agg_a = hi(S, Wf0_ref[...]) + cnt * mbf_ref[0]
    agg_b = (hi(S, Wf1_ref[...]) + cnt * mbf_ref[1]) / jnp.maximum(cnt, 1.0)
    x = agg_a + agg_b

    def bn(v, gm, bt):
        m = jnp.mean(v, axis=0)
        q = jnp.mean(v * v, axis=0)
        s = gm * lax.rsqrt(q - m * m + _EPS)
        return (v - m) * s + bt

    def mm(a, W_ref):
        return jnp.dot(a.astype(_BF16), W_ref[...],
                       preferred_element_type=_F32)

    x = mm(x, uW0_ref) + P_ref[0]
    x = mm(jnp.tanh(bn(x, P_ref[3], P_ref[4])), uWa_ref) + P_ref[1]
    x = mm(jnp.tanh(bn(x, P_ref[3], P_ref[4])), uWa_ref) + P_ref[1]
    x = mm(jnp.tanh(bn(x, P_ref[5], P_ref[6])), uWb_ref) + P_ref[2]
    x = jnp.tanh(bn(x, P_ref[7], P_ref[8]))
    out_ref[...] = mm(x, Wpred_ref) + P_ref[9]


def _edge_stats_pass(x, W16, block_e):
    E = x.shape[0]
    return pl.pallas_call(
        _edge_stats_body,
        grid=(E // block_e,),
        in_specs=[
            pl.BlockSpec((block_e, 128), lambda i: (i, 0)),
            pl.BlockSpec((128, 128), lambda i: (0, 0)),
        ],
        out_specs=[
            pl.BlockSpec((8, 128), lambda i: (0, 0)),
            pl.BlockSpec((8, 128), lambda i: (0, 0)),
        ],
        out_shape=[
            jax.ShapeDtypeStruct((8, 128), _F32),
            jax.ShapeDtypeStruct((8, 128), _F32),
        ],
        compiler_params=pltpu.CompilerParams(
            dimension_semantics=("arbitrary",)),
    )(x, W16)


def _edge_tmm_pass(x, params, W16, block_e):
    E = x.shape[0]
    return pl.pallas_call(
        _edge_tmm_body,
        grid=(E // block_e,),
        in_specs=[
            pl.BlockSpec((block_e, 128), lambda i: (i, 0)),
            pl.BlockSpec((8, 128), lambda i: (0, 0)),
            pl.BlockSpec((128, 128), lambda i: (0, 0)),
        ],
        out_specs=[
            pl.BlockSpec((block_e, 128), lambda i: (i, 0)),
            pl.BlockSpec((8, 128), lambda i: (0, 0)),
            pl.BlockSpec((128, 128), lambda i: (0, 0)),
        ],
        out_shape=[
            jax.ShapeDtypeStruct((E, 128), _BF16),
            jax.ShapeDtypeStruct((8, 128), _F32),
            jax.ShapeDtypeStruct((128, 128), _F32),
        ],
        compiler_params=pltpu.CompilerParams(
            dimension_semantics=("arbitrary",)),
    )(x, params, W16)


def _edge_last_pass(x, params, W16, block_e):
    E = x.shape[0]
    return pl.pallas_call(
        _edge_last_body,
        grid=(E // block_e,),
        in_specs=[
            pl.BlockSpec((block_e, 128), lambda i: (i, 0)),
            pl.BlockSpec((8, 128), lambda i: (0, 0)),
            pl.BlockSpec((128, 128), lambda i: (0, 0)),
        ],
        out_specs=pl.BlockSpec((block_e, 128), lambda i: (i, 0)),
        out_shape=jax.ShapeDtypeStruct((E, 128), _F32),
        compiler_params=pltpu.CompilerParams(
            dimension_semantics=("arbitrary",)),
    )(x, params, W16)


def _sc_gather_diff(h, src2d, dst2d):
    NCHT, CH = src2d.shape
    E = NCHT * CH
    EW = E // _NW
    NCH = EW // CH
    mesh = plsc.VectorSubcoreMesh(core_axis_name="c", subcore_axis_name="s")

    @functools.partial(
        pl.kernel,
        out_type=jax.ShapeDtypeStruct((E, 128), _F32),
        mesh=mesh,
        scratch_types=[
            pltpu.VMEM((NCH, CH), jnp.int32),
            pltpu.VMEM((NCH, CH), jnp.int32),
            pltpu.VMEM((CH, 128), _F32),
            pltpu.VMEM((CH, 128), _F32),
            pltpu.VMEM((CH, 128), _F32),
            pltpu.VMEM((CH, 128), _F32),
            pltpu.VMEM((CH, 128), _F32),
            pltpu.VMEM((CH, 128), _F32),
            pltpu.VMEM((CH, 128), _F32),
            pltpu.VMEM((CH, 128), _F32),
            pltpu.VMEM((CH, 128), _F32),
            pltpu.SemaphoreType.DMA,
            pltpu.SemaphoreType.DMA,
            pltpu.SemaphoreType.DMA,
            pltpu.SemaphoreType.DMA,
            pltpu.SemaphoreType.DMA,
            pltpu.SemaphoreType.DMA,
        ],
        compiler_params=pltpu.CompilerParams(use_tc_tiling_on_sc=False),
    )
    def k(h_hbm, src_hbm, dst_hbm, d_hbm, isrc, idst, gs0, gs1, gs2,
          gd0, gd1, gd2, db0, db1, db2, semg0, semg1, semg2,
          semw0, semw1, semw2):
        gs = (gs0, gs1, gs2)
        gd = (gd0, gd1, gd2)
        db = (db0, db1, db2)
        semg = (semg0, semg1, semg2)
        semw = (semw0, semw1, semw2)
        wid = lax.axis_index("s") * _NC + lax.axis_index("c")
        base = wid * EW
        row0 = wid * NCH

        pltpu.sync_copy(src_hbm.at[pl.ds(row0, NCH)], isrc)
        pltpu.sync_copy(dst_hbm.at[pl.ds(row0, NCH)], idst)

        def start_gathers(ci, s):
            pltpu.async_copy(h_hbm.at[isrc.at[ci]], gs[s], semg[s])
            pltpu.async_copy(h_hbm.at[idst.at[ci]], gd[s], semg[s])

        def wait_gathers(s):
            pltpu.make_async_copy(h_hbm.at[isrc.at[0]], gs[s], semg[s]).wait()
            pltpu.make_async_copy(h_hbm.at[idst.at[0]], gd[s], semg[s]).wait()

        def wait_write(s, ci):
            off = pl.multiple_of(base + ci * CH, 8)
            pltpu.make_async_copy(db[s], d_hbm.at[pl.ds(off, CH)],
                                  semw[s]).wait()

        start_gathers(0, 0)
        start_gathers(1, 1)

        def tri_body(t, _):
            for b in range(3):
                ci = 3 * t + b
                p = (b + 2) % 3

                @pl.when(ci < NCH)
                def _work():
                    wait_gathers(b)

                    @pl.when(ci >= 3)
                    def _ww():
                        wait_write(b, ci - 3)

                    def row_body(r, _):
                        for grp in range(8):
                            sl = pl.ds(grp * 16, 16)
                            db[b][r, sl] = gd[b][r, sl] - gs[b][r, sl]
                        return 0

                    lax.fori_loop(0, CH, row_body, 0)
                    off = pl.multiple_of(base + ci * CH, 8)
                    pltpu.async_copy(db[b], d_hbm.at[pl.ds(off, CH)], semw[b])

                    @pl.when(ci + 2 < NCH)
                    def _pf():
                        start_gathers(ci + 2, p)

            return 0

        lax.fori_loop(0, (NCH + 2) // 3, tri_body, 0)
        for last in range(max(NCH - 3, 0), NCH):
            wait_write(last % 3, last)

    return k(h, src2d, dst2d)


def _sc_scatter_add(t3, dst2d, n_nodes):
    NCHT, CH = dst2d.shape
    E = NCHT * CH
    EW = E // _NS
    NCH = EW // CH
    H = n_nodes // _NC
    HP = H + 128
    RPT = H // _NS // 8 * 8
    RLAST = H - (_NS - 1) * RPT
    CPT = RPT // 8
    CLAST = RLAST // 8
    mesh = plsc.VectorSubcoreMesh(core_axis_name="c", subcore_axis_name="s")

    @functools.partial(
        pl.kernel,
        out_type=[
            jax.ShapeDtypeStruct((n_nodes, 128), _F32),
            jax.ShapeDtypeStruct((n_nodes // 8, 128), _F32),
        ],
        mesh=mesh,
        scratch_types=[
            pltpu.VMEM((NCH, CH), jnp.int32),
            pltpu.VMEM((CH, 128), _F32),
            pltpu.VMEM((CH, 128), _F32),
            pltpu.VMEM((CH, 128), _F32),
            pltpu.VMEM((CH, 16), _F32),
            pltpu.VMEM((RLAST, 16), _F32),
            pltpu.VMEM((CLAST, 128), _F32),
            pltpu.VMEM_SHARED((HP, 128), _F32),
            pltpu.VMEM_SHARED((HP, 16), _F32),
            pltpu.SemaphoreType.DMA,
            pltpu.SemaphoreType.DMA,
            pltpu.SemaphoreType.DMA,
            pltpu.SemaphoreType.DMA,
            pltpu.SemaphoreType.DMA,
            pltpu.SemaphoreType.DMA,
            pltpu.SemaphoreType.DMA,
            pltpu.SemaphoreType.DMA,
            pltpu.SemaphoreType.DMA,
        ],
        compiler_params=pltpu.CompilerParams(use_tc_tiling_on_sc=False),
    )
    def k(t3_hbm, dst_hbm, zS_hbm, zC_hbm, outS, outC,
          idxa, tbuf0, tbuf1, tbuf2, onesv, bC, bC2, S_sh, C_sh,
          semt0, semt1, semt2, sems0, sems1, sems2, semo0, semo1, semo2):
        tbuf = (tbuf0, tbuf1, tbuf2)
        semt = (semt0, semt1, semt2)
        sems = (sems0, sems1, sems2)
        semo = (semo0, semo1, semo2)
        cid = lax.axis_index("c")
        sid = lax.axis_index("s")
        node0 = cid * H
        ro = sid * RPT
        row0 = sid * NCH

        one = jnp.full((16,), 1.0, _F32)

        def ones_body(r, _):
            onesv[r, :] = one
            return 0

        lax.fori_loop(0, CH, ones_body, 0)

        @pl.when(sid == 0)
        def _z0():
            pltpu.sync_copy(zS_hbm, S_sh)
            pltpu.sync_copy(zC_hbm, C_sh)

        pltpu.sync_copy(dst_hbm.at[pl.ds(row0, NCH)], idxa)

        def adj_body(r, _):
            for j in range(CH // 16):
                sl = pl.ds(j * 16, 16)
                local = idxa[r, sl] - node0
                ok = (local >= 0) & (local < H)
                idxa[r, sl] = jnp.where(ok, local, H)
            return 0

        lax.fori_loop(0, NCH, adj_body, 0)
        plsc.subcore_barrier()

        def load_t3(ci, s):
            off = pl.multiple_of(sid * EW + ci * CH, 8)
            pltpu.async_copy(t3_hbm.at[pl.ds(off, CH)], tbuf[s], semt[s])

        def wait_load(s):
            pltpu.make_async_copy(t3_hbm.at[pl.ds(0, CH)], tbuf[s],
                                  semt[s]).wait()

        def wait_scats(s):
            pltpu.make_async_copy(tbuf[s], S_sh.at[pl.ds(0, CH)],
                                  sems[s]).wait()
            pltpu.make_async_copy(onesv, C_sh.at[pl.ds(0, CH)],
                                  semo[s]).wait()

        load_t3(0, 0)
        load_t3(1, 1)

        def tri_body(t, _):
            for b in range(3):
                ci = 3 * t + b
                p = (b + 2) % 3

                @pl.when(ci < NCH)
                def _work():
                    wait_load(b)
                    pltpu.async_copy(tbuf[b], S_sh.at[idxa.at[ci]], sems[b],
                                     add=True)
                    pltpu.async_copy(onesv, C_sh.at[idxa.at[ci]], semo[b],
                                     add=True)

                    @pl.when(ci + 2 < NCH)
                    def _pf():
                        @pl.when(ci >= 1)
                        def _ws():
                            wait_scats(p)

                        load_t3(ci + 2, p)

            return 0

        lax.fori_loop(0, (NCH + 2) // 3, tri_body, 0)
        for last in range(max(NCH - 3, 0), NCH):
            wait_scats(last % 3)
        plsc.subcore_barrier()

        @pl.when(sid != _NS - 1)
        def _r0():
            pltpu.sync_copy(S_sh.at[pl.ds(ro, RPT)],
                            outS.at[pl.ds(node0 + ro, RPT)])
            pltpu.sync_copy(C_sh.at[pl.ds(ro, RPT)], bC.at[pl.ds(0, RPT)])

        @pl.when(sid == _NS - 1)
        def _r1():
            pltpu.sync_copy(S_sh.at[pl.ds(ro, RLAST)],
                            outS.at[pl.ds(node0 + ro, RLAST)])
            pltpu.sync_copy(C_sh.at[pl.ds(ro, RLAST)], bC)

        def repack_body(r, _):
            bC2[r // 8, pl.ds((r % 8) * 16, 16)] = bC[r, :]
            return 0

        lax.fori_loop(0, RLAST, repack_body, 0)
        co = cid * (H // 8) + sid * CPT

        @pl.when(sid != _NS - 1)
        def _c0():
            pltpu.sync_copy(bC2.at[pl.ds(0, CPT)], outC.at[pl.ds(co, CPT)])

        @pl.when(sid == _NS - 1)
        def _c1():
            pltpu.sync_copy(bC2, outC.at[pl.ds(co, CLAST)])

    zS = jnp.zeros((HP, 128), _F32)
    zC = jnp.zeros((HP, 16), _F32)
    return k(t3, dst2d, zS, zC)


def _affine_from_stats(ps, pq, count, gamma, beta):
    m = jnp.sum(ps, 0) / count
    var = jnp.sum(pq, 0) / count - m * m
    scale = gamma * lax.rsqrt(var + _EPS)
    return scale, beta - m * scale


def _affine_from_gram(cs, G, Wr, badd, count, gamma, beta):
    hi = functools.partial(jnp.dot, precision=lax.Precision.HIGHEST)
    mu = hi(jnp.sum(cs, 0) / count, Wr)
    T = hi(G / count, Wr)
    ex2 = jnp.sum(Wr * T, axis=0)
    var = ex2 - mu * mu
    scale = gamma * lax.rsqrt(var + _EPS)
    return scale, beta - (mu + badd) * scale


def kernel(pos, vel, edge_index, W_in, b_in, mW0, mb0, mWa, mba, mWf, mbf,
           mg_a, mbe_a, mg_b, mbe_b, uW0, ub0, uWa, uba, uWb, ubb,
           ug_a, ube_a, ug_b, ube_b, ug_c, ube_c, W_pred, b_pred):
    N = pos.shape[0]
    E = edge_index.shape[1]
    src = edge_index[0]
    dst = edge_index[1]
    BE = 2560
    fE = jnp.float32(E)

    pv = jnp.zeros((N, 128), _F32)
    pv = pv.at[:, 0:2].set(pos).at[:, 2:4].set(vel)
    Wp = jnp.zeros((128, 128), _F32).at[0:4, :].set(W_in)
    h = pl.pallas_call(
        _node_prologue_body,
        out_shape=jax.ShapeDtypeStruct((N, 128), _F32),
    )(pv, Wp, b_in.reshape(1, 128))

    src2d = src.reshape(E // _CH, _CH)
    dst2d = dst.reshape(E // _CH, _CH)
    d = _sc_gather_diff(h, src2d, dst2d)

    def pack_p(scale, bias, bout):
        p = jnp.zeros((8, 128), _F32)
        return p.at[0].set(scale).at[1].set(bias).at[2].set(bout)

    mW0_16 = mW0.astype(_BF16)
    mWa_16 = mWa.astype(_BF16)
    zeros128 = jnp.zeros((128,), _F32)

    ps1, pq1 = _edge_stats_pass(d, mW0_16, BE)
    s1, b1 = _affine_from_stats(ps1, pq1, fE, mg_a, mbe_a)

    t1b, cs2, G2 = _edge_tmm_pass(d, pack_p(s1, b1, zeros128), mW0_16, BE)
    s2, b2 = _affine_from_gram(cs2, G2, _bfr(mWa), mba, fE, mg_a, mbe_a)

    t2b, cs3, G3 = _edge_tmm_pass(t1b, pack_p(s2, b2, mba), mWa_16, BE)
    s3, b3 = _affine_from_gram(cs3, G3, _bfr(mWa), mba, fE, mg_b, mbe_b)

    t3 = _edge_last_pass(t2b, pack_p(s3, b3, mba), mWa_16, BE)

    S_seg, C_raw = _sc_scatter_add(t3, dst2d, N)
    cnt = jnp.broadcast_to(C_raw.reshape(N, 16)[:, 0:1], (N, 8))

    P = jnp.zeros((16, 128), _F32)
    for i, v in enumerate([ub0, uba, ubb, ug_a, ube_a, ug_b, ube_b,
                           ug_c, ube_c]):
        P = P.at[i].set(v)
    P = P.at[9, 0:2].set(b_pred)
    Wpred16 = jnp.zeros((128, 128), _F32).at[:, 0:2].set(W_pred).astype(_BF16)
    out = pl.pallas_call(
        _finale_body,
        out_shape=jax.ShapeDtypeStruct((N, 128), _F32),
    )(S_seg, cnt, _bfr(mWf[:, :128]), _bfr(mWf[:, 128:]),
      mbf.reshape(2, 128), uW0.astype(_BF16), uWa.astype(_BF16),
      uWb.astype(_BF16), Wpred16, P)
    return out[:, 0:2]


# speedup vs baseline: 1.6362x; 1.2144x over previous
import functools

import jax
import jax.numpy as jnp
from jax import lax
from jax.experimental import pallas as pl
from jax.experimental.pallas import tpu as pltpu
from jax.experimental.pallas import tpu_sc as plsc

_EPS = 1e-5
_NC = 2
_NS = 16
_NW = _NC * _NS
_CH = 80

_F32 = jnp.float32
_BF16 = jnp.bfloat16


def _bfr(x):
    return x.astype(_BF16).astype(_F32)


def _node_prologue_body(pv_ref, Wp_ref, bp_ref, h_ref):
    h = jnp.dot(pv_ref[...].astype(_BF16), Wp_ref[...].astype(_BF16),
                preferred_element_type=_F32)
    h_ref[...] = h + bp_ref[0]


def _stats_accum(i, y, ps_ref, pq_ref):
    yr = y.reshape(-1, 8, 128)
    ps = jnp.sum(yr, axis=0)
    pq = jnp.sum(yr * yr, axis=0)

    @pl.when(i == 0)
    def _init():
        ps_ref[...] = ps
        pq_ref[...] = pq

    @pl.when(i != 0)
    def _acc():
        ps_ref[...] = ps_ref[...] + ps
        pq_ref[...] = pq_ref[...] + pq


def _edge_stats_body(x_ref, W_ref, ps_ref, pq_ref):
    y = jnp.dot(x_ref[...].astype(_BF16), W_ref[...],
                preferred_element_type=_F32)
    _stats_accum(pl.program_id(0), y, ps_ref, pq_ref)


def _edge_tmm_body(x_ref, p_ref, W_ref, t_ref, cs_ref, G_ref):
    i = pl.program_id(0)
    xv = x_ref[...]
    if xv.dtype != _BF16:
        xv = xv.astype(_BF16)
    x = jnp.dot(xv, W_ref[...], preferred_element_type=_F32) + p_ref[2]
    t16 = jnp.tanh(x * p_ref[0] + p_ref[1]).astype(_BF16)
    t_ref[...] = t16
    tf = t16.astype(_F32)
    yr = tf.reshape(-1, 8, 128)
    cs = jnp.sum(yr, axis=0)
    G = lax.dot_general(t16, t16, (((0,), (0,)), ((), ())),
                        preferred_element_type=_F32)

    @pl.when(i == 0)
    def _init():
        cs_ref[...] = cs
        G_ref[...] = G

    @pl.when(i != 0)
    def _acc():
        cs_ref[...] = cs_ref[...] + cs
        G_ref[...] = G_ref[...] + G


def _edge_last_body(x_ref, p_ref, W_ref, t_ref):
    x = jnp.dot(x_ref[...], W_ref[...],
                preferred_element_type=_F32) + p_ref[2]
    t_ref[...] = _bfr(jnp.tanh(x * p_ref[0] + p_ref[1]))


def _finale_body(S_ref, C_ref, Wf0_ref, Wf1_ref, mbf_ref, uW0_ref, uWa_ref,
                 uWb_ref, Wpred_ref, P_ref, out_ref):
    S = S_ref[...]
    cnt = C_ref[:, 0:1]

    hi = functools.partial(jnp.dot, preferred_element_type=_F32,
                           precision=lax.Precision.HIGHEST)
    agg_a = hi(S, Wf0_ref[...]) + cnt * mbf_ref[0]
    agg_b = (hi(S, Wf1_ref[...]) + cnt * mbf_ref[1]) / jnp.maximum(cnt, 1.0)
    x = agg_a + agg_b

    def bn(v, gm, bt):
        m = jnp.mean(v, axis=0)
        q = jnp.mean(v * v, axis=0)
        s = gm * lax.rsqrt(q - m * m + _EPS)
        return (v - m) * s + bt

    def mm(a, W_ref):
        return jnp.dot(a.astype(_BF16), W_ref[...],
                       preferred_element_type=_F32)

    x = mm(x, uW0_ref) + P_ref[0]
    x = mm(jnp.tanh(bn(x, P_ref[3], P_ref[4])), uWa_ref) + P_ref[1]
    x = mm(jnp.tanh(bn(x, P_ref[3], P_ref[4])), uWa_ref) + P_ref[1]
    x = mm(jnp.tanh(bn(x, P_ref[5], P_ref[6])), uWb_ref) + P_ref[2]
    x = jnp.tanh(bn(x, P_ref[7], P_ref[8]))
    out_ref[...] = mm(x, Wpred_ref) + P_ref[9]


def _edge_stats_pass(x, W16, block_e):
    E = x.shape[0]
    return pl.pallas_call(
        _edge_stats_body,
        grid=(E // block_e,),
        in_specs=[
            pl.BlockSpec((block_e, 128), lambda i: (i, 0)),
            pl.BlockSpec((128, 128), lambda i: (0, 0)),
        ],
        out_specs=[
            pl.BlockSpec((8, 128), lambda i: (0, 0)),
            pl.BlockSpec((8, 128), lambda i: (0, 0)),
        ],
        out_shape=[
            jax.ShapeDtypeStruct((8, 128), _F32),
            jax.ShapeDtypeStruct((8, 128), _F32),
        ],
        compiler_params=pltpu.CompilerParams(
            dimension_semantics=("arbitrary",)),
    )(x, W16)


def _edge_tmm_pass(x, params, W16, block_e):
    E = x.shape[0]
    return pl.pallas_call(
        _edge_tmm_body,
        grid=(E // block_e,),
        in_specs=[
            pl.BlockSpec((block_e, 128), lambda i: (i, 0)),
            pl.BlockSpec((8, 128), lambda i: (0, 0)),
            pl.BlockSpec((128, 128), lambda i: (0, 0)),
        ],
        out_specs=[
            pl.BlockSpec((block_e, 128), lambda i: (i, 0)),
            pl.BlockSpec((8, 128), lambda i: (0, 0)),
            pl.BlockSpec((128, 128), lambda i: (0, 0)),
        ],
        out_shape=[
            jax.ShapeDtypeStruct((E, 128), _BF16),
            jax.ShapeDtypeStruct((8, 128), _F32),
            jax.ShapeDtypeStruct((128, 128), _F32),
        ],
        compiler_params=pltpu.CompilerParams(
            dimension_semantics=("arbitrary",)),
    )(x, params, W16)


def _edge_last_pass(x, params, W16, block_e):
    E = x.shape[0]
    return pl.pallas_call(
        _edge_last_body,
        grid=(E // block_e,),
        in_specs=[
            pl.BlockSpec((block_e, 128), lambda i: (i, 0)),
            pl.BlockSpec((8, 128), lambda i: (0, 0)),
            pl.BlockSpec((128, 128), lambda i: (0, 0)),
        ],
        out_specs=pl.BlockSpec((block_e, 128), lambda i: (i, 0)),
        out_shape=jax.ShapeDtypeStruct((E, 128), _F32),
        compiler_params=pltpu.CompilerParams(
            dimension_semantics=("arbitrary",)),
    )(x, params, W16)


def _sc_gather_diff(h, src2d, dst2d):
    NCHT, CH = src2d.shape
    E = NCHT * CH
    EW = E // _NW
    NCH = EW // CH
    mesh = plsc.VectorSubcoreMesh(core_axis_name="c", subcore_axis_name="s")

    @functools.partial(
        pl.kernel,
        out_type=jax.ShapeDtypeStruct((E, 128), _F32),
        mesh=mesh,
        scratch_types=[
            pltpu.VMEM((NCH, CH), jnp.int32),
            pltpu.VMEM((NCH, CH), jnp.int32),
            pltpu.VMEM((CH, 128), _F32),
            pltpu.VMEM((CH, 128), _F32),
            pltpu.VMEM((CH, 128), _F32),
            pltpu.VMEM((CH, 128), _F32),
            pltpu.VMEM((CH, 128), _F32),
            pltpu.VMEM((CH, 128), _F32),
            pltpu.VMEM((CH, 128), _F32),
            pltpu.VMEM((CH, 128), _F32),
            pltpu.VMEM((CH, 128), _F32),
            pltpu.SemaphoreType.DMA,
            pltpu.SemaphoreType.DMA,
            pltpu.SemaphoreType.DMA,
            pltpu.SemaphoreType.DMA,
            pltpu.SemaphoreType.DMA,
            pltpu.SemaphoreType.DMA,
        ],
        compiler_params=pltpu.CompilerParams(use_tc_tiling_on_sc=False),
    )
    def k(h_hbm, src_hbm, dst_hbm, d_hbm, isrc, idst, gs0, gs1, gs2,
          gd0, gd1, gd2, db0, db1, db2, semg0, semg1, semg2,
          semw0, semw1, semw2):
        gs = (gs0, gs1, gs2)
        gd = (gd0, gd1, gd2)
        db = (db0, db1, db2)
        semg = (semg0, semg1, semg2)
        semw = (semw0, semw1, semw2)
        wid = lax.axis_index("s") * _NC + lax.axis_index("c")
        base = wid * EW
        row0 = wid * NCH

        pltpu.sync_copy(src_hbm.at[pl.ds(row0, NCH)], isrc)
        pltpu.sync_copy(dst_hbm.at[pl.ds(row0, NCH)], idst)

        def start_gathers(ci, s):
            pltpu.async_copy(h_hbm.at[isrc.at[ci]], gs[s], semg[s])
            pltpu.async_copy(h_hbm.at[idst.at[ci]], gd[s], semg[s])

        def wait_gathers(s):
            pltpu.make_async_copy(h_hbm.at[isrc.at[0]], gs[s], semg[s]).wait()
            pltpu.make_async_copy(h_hbm.at[idst.at[0]], gd[s], semg[s]).wait()

        def wait_write(s, ci):
            off = pl.multiple_of(base + ci * CH, 8)
            pltpu.make_async_copy(db[s], d_hbm.at[pl.ds(off, CH)],
                                  semw[s]).wait()

        start_gathers(0, 0)
        start_gathers(1, 1)

        def tri_body(t, _):
            for b in range(3):
                ci = 3 * t + b
                p = (b + 2) % 3

                @pl.when(ci < NCH)
                def _work():
                    wait_gathers(b)

                    @pl.when(ci >= 3)
                    def _ww():
                        wait_write(b, ci - 3)

                    def row_body(r, _):
                        for grp in range(8):
                            sl = pl.ds(grp * 16, 16)
                            db[b][r, sl] = gd[b][r, sl] - gs[b][r, sl]
                        return 0

                    lax.fori_loop(0, CH, row_body, 0)
                    off = pl.multiple_of(base + ci * CH, 8)
                    pltpu.async_copy(db[b], d_hbm.at[pl.ds(off, CH)], semw[b])

                    @pl.when(ci + 2 < NCH)
                    def _pf():
                        start_gathers(ci + 2, p)

            return 0

        lax.fori_loop(0, (NCH + 2) // 3, tri_body, 0)
        for last in range(max(NCH - 3, 0), NCH):
            wait_write(last % 3, last)

    return k(h, src2d, dst2d)


def _sc_scatter_add(t3, dst2d, n_nodes):
    NCHT, CH = dst2d.shape
    E = NCHT * CH
    EW = E // _NS
    NCH = EW // CH
    H = n_nodes // _NC
    HP = H + 128
    RPT = H // _NS // 8 * 8
    RLAST = H - (_NS - 1) * RPT
    CPT = RPT // 8
    CLAST = RLAST // 8
    mesh = plsc.VectorSubcoreMesh(core_axis_name="c", subcore_axis_name="s")

    @functools.partial(
        pl.kernel,
        out_type=[
            jax.ShapeDtypeStruct((n_nodes, 128), _F32),
            jax.ShapeDtypeStruct((n_nodes // 8, 128), _F32),
        ],
        mesh=mesh,
        scratch_types=[
            pltpu.VMEM((NCH, CH), jnp.int32),
            pltpu.VMEM((CH, 128), _F32),
            pltpu.VMEM((CH, 128), _F32),
            pltpu.VMEM((CH, 128), _F32),
            pltpu.VMEM((CH, 16), _F32),
            pltpu.VMEM((RLAST, 16), _F32),
            pltpu.VMEM((CLAST, 128), _F32),
            pltpu.VMEM_SHARED((HP, 128), _F32),
            pltpu.VMEM_SHARED((HP, 16), _F32),
            pltpu.SemaphoreType.DMA,
            pltpu.SemaphoreType.DMA,
            pltpu.SemaphoreType.DMA,
            pltpu.SemaphoreType.DMA,
            pltpu.SemaphoreType.DMA,
            pltpu.SemaphoreType.DMA,
            pltpu.SemaphoreType.DMA,
            pltpu.SemaphoreType.DMA,
            pltpu.SemaphoreType.DMA,
        ],
        compiler_params=pltpu.CompilerParams(use_tc_tiling_on_sc=False),
    )
    def k(t3_hbm, dst_hbm, zS_hbm, zC_hbm, outS, outC,
          idxa, tbuf0, tbuf1, tbuf2, onesv, bC, bC2, S_sh, C_sh,
          semt0, semt1, semt2, sems0, sems1, sems2, semo0, semo1, semo2):
        tbuf = (tbuf0, tbuf1, tbuf2)
        semt = (semt0, semt1, semt2)
        sems = (sems0, sems1, sems2)
        semo = (semo0, semo1, semo2)
        cid = lax.axis_index("c")
        sid = lax.axis_index("s")
        node0 = cid * H
        ro = sid * RPT
        row0 = sid * NCH

        one = jnp.full((16,), 1.0, _F32)

        def ones_body(r, _):
            onesv[r, :] = one
            return 0

        lax.fori_loop(0, CH, ones_body, 0)

        @pl.when(sid == 0)
        def _z0():
            pltpu.sync_copy(zS_hbm, S_sh)
            pltpu.sync_copy(zC_hbm, C_sh)

        pltpu.sync_copy(dst_hbm.at[pl.ds(row0, NCH)], idxa)

        def adj_body(r, _):
            for j in range(CH // 16):
                sl = pl.ds(j * 16, 16)
                local = idxa[r, sl] - node0
                ok = (local >= 0) & (local < H)
                idxa[r, sl] = jnp.where(ok, local, H)
            return 0

        lax.fori_loop(0, NCH, adj_body, 0)
        plsc.subcore_barrier()

        def load_t3(ci, s):
            off = pl.multiple_of(sid * EW + ci * CH, 8)
            pltpu.async_copy(t3_hbm.at[pl.ds(off, CH)], tbuf[s], semt[s])

        def wait_load(s):
            pltpu.make_async_copy(t3_hbm.at[pl.ds(0, CH)], tbuf[s],
                                  semt[s]).wait()

        def wait_scats(s):
            pltpu.make_async_copy(tbuf[s], S_sh.at[pl.ds(0, CH)],
                                  sems[s]).wait()
            pltpu.make_async_copy(onesv, C_sh.at[pl.ds(0, CH)],
                                  semo[s]).wait()

        load_t3(0, 0)
        load_t3(1, 1)

        def tri_body(t, _):
            for b in range(3):
                ci = 3 * t + b
                p = (b + 2) % 3

                @pl.when(ci < NCH)
                def _work():
                    wait_load(b)
                    pltpu.async_copy(tbuf[b], S_sh.at[idxa.at[ci]], sems[b],
                                     add=True)
                    pltpu.async_copy(onesv, C_sh.at[idxa.at[ci]], semo[b],
                                     add=True)

                    @pl.when(ci + 2 < NCH)
                    def _pf():
                        @pl.when(ci >= 1)
                        def _ws():
                            wait_scats(p)

                        load_t3(ci + 2, p)

            return 0

        lax.fori_loop(0, (NCH + 2) // 3, tri_body, 0)
        for last in range(max(NCH - 3, 0), NCH):
            wait_scats(last % 3)
        plsc.subcore_barrier()

        @pl.when(sid != _NS - 1)
        def _r0():
            pltpu.sync_copy(S_sh.at[pl.ds(ro, RPT)],
                            outS.at[pl.ds(node0 + ro, RPT)])
            pltpu.sync_copy(C_sh.at[pl.ds(ro, RPT)], bC.at[pl.ds(0, RPT)])

        @pl.when(sid == _NS - 1)
        def _r1():
            pltpu.sync_copy(S_sh.at[pl.ds(ro, RLAST)],
                            outS.at[pl.ds(node0 + ro, RLAST)])
            pltpu.sync_copy(C_sh.at[pl.ds(ro, RLAST)], bC)

        def repack_body(r, _):
            bC2[r // 8, pl.ds((r % 8) * 16, 16)] = bC[r, :]
            return 0

        lax.fori_loop(0, RLAST, repack_body, 0)
        co = cid * (H // 8) + sid * CPT

        @pl.when(sid != _NS - 1)
        def _c0():
            pltpu.sync_copy(bC2.at[pl.ds(0, CPT)], outC.at[pl.ds(co, CPT)])

        @pl.when(sid == _NS - 1)
        def _c1():
            pltpu.sync_copy(bC2, outC.at[pl.ds(co, CLAST)])

    zS = jnp.zeros((HP, 128), _F32)
    zC = jnp.zeros((HP, 16), _F32)
    return k(t3, dst2d, zS, zC)


def _affine_from_stats(ps, pq, count, gamma, beta):
    m = jnp.sum(ps, 0) / count
    var = jnp.sum(pq, 0) / count - m * m
    scale = gamma * lax.rsqrt(var + _EPS)
    return scale, beta - m * scale


def _affine_from_gram(cs, G, Wr, badd, count, gamma, beta):
    hi = functools.partial(jnp.dot, precision=lax.Precision.HIGHEST)
    mu = hi(jnp.sum(cs, 0) / count, Wr)
    T = hi(G / count, Wr)
    ex2 = jnp.sum(Wr * T, axis=0)
    var = ex2 - mu * mu
    scale = gamma * lax.rsqrt(var + _EPS)
    return scale, beta - (mu + badd) * scale


def kernel(pos, vel, edge_index, W_in, b_in, mW0, mb0, mWa, mba, mWf, mbf,
           mg_a, mbe_a, mg_b, mbe_b, uW0, ub0, uWa, uba, uWb, ubb,
           ug_a, ube_a, ug_b, ube_b, ug_c, ube_c, W_pred, b_pred):
    N = pos.shape[0]
    E = edge_index.shape[1]
    src = edge_index[0]
    dst = edge_index[1]
    BE = 8000
    fE = jnp.float32(E)

    pv = jnp.zeros((N, 128), _F32)
    pv = pv.at[:, 0:2].set(pos).at[:, 2:4].set(vel)
    Wp = jnp.zeros((128, 128), _F32).at[0:4, :].set(W_in)
    h = pl.pallas_call(
        _node_prologue_body,
        out_shape=jax.ShapeDtypeStruct((N, 128), _F32),
    )(pv, Wp, b_in.reshape(1, 128))

    src2d = src.reshape(E // _CH, _CH)
    dst2d = dst.reshape(E // _CH, _CH)
    d = _sc_gather_diff(h, src2d, dst2d)

    def pack_p(scale, bias, bout):
        p = jnp.zeros((8, 128), _F32)
        return p.at[0].set(scale).at[1].set(bias).at[2].set(bout)

    mW0_16 = mW0.astype(_BF16)
    mWa_16 = mWa.astype(_BF16)
    zeros128 = jnp.zeros((128,), _F32)

    ps1, pq1 = _edge_stats_pass(d, mW0_16, BE)
    s1, b1 = _affine_from_stats(ps1, pq1, fE, mg_a, mbe_a)

    t1b, cs2, G2 = _edge_tmm_pass(d, pack_p(s1, b1, zeros128), mW0_16, BE)
    s2, b2 = _affine_from_gram(cs2, G2, _bfr(mWa), mba, fE, mg_a, mbe_a)

    t2b, cs3, G3 = _edge_tmm_pass(t1b, pack_p(s2, b2, mba), mWa_16, BE)
    s3, b3 = _affine_from_gram(cs3, G3, _bfr(mWa), mba, fE, mg_b, mbe_b)

    t3 = _edge_last_pass(t2b, pack_p(s3, b3, mba), mWa_16, BE)

    S_seg, C_raw = _sc_scatter_add(t3, dst2d, N)
    cnt = jnp.broadcast_to(C_raw.reshape(N, 16)[:, 0:1], (N, 8))

    P = jnp.zeros((16, 128), _F32)
    for i, v in enumerate([ub0, uba, ubb, ug_a, ube_a, ug_b, ube_b,
                           ug_c, ube_c]):
        P = P.at[i].set(v)
    P = P.at[9, 0:2].set(b_pred)
    Wpred16 = jnp.zeros((128, 128), _F32).at[:, 0:2].set(W_pred).astype(_BF16)
    out = pl.pallas_call(
        _finale_body,
        out_shape=jax.ShapeDtypeStruct((N, 128), _F32),
    )(S_seg, cnt, _bfr(mWf[:, :128]), _bfr(mWf[:, 128:]),
      mbf.reshape(2, 128), uW0.astype(_BF16), uWa.astype(_BF16),
      uWb.astype(_BF16), Wpred16, P)
    return out[:, 0:2]


# speedup vs baseline: 1.7234x; 1.0533x over previous
import functools

import jax
import jax.numpy as jnp
from jax import lax
from jax.experimental import pallas as pl
from jax.experimental.pallas import tpu as pltpu
from jax.experimental.pallas import tpu_sc as plsc

_EPS = 1e-5
_NC = 2
_NS = 16
_NW = _NC * _NS
_CH = 80

_F32 = jnp.float32
_BF16 = jnp.bfloat16


def _bfr(x):
    return x.astype(_BF16).astype(_F32)


def _node_prologue_body(pv_ref, Wp_ref, bp_ref, h_ref):
    h = jnp.dot(pv_ref[...].astype(_BF16), Wp_ref[...].astype(_BF16),
                preferred_element_type=_F32)
    h_ref[...] = h + bp_ref[0]


def _stats_accum(i, y, ps_ref, pq_ref):
    yr = y.reshape(-1, 8, 128)
    ps = jnp.sum(yr, axis=0)
    pq = jnp.sum(yr * yr, axis=0)

    @pl.when(i == 0)
    def _init():
        ps_ref[...] = ps
        pq_ref[...] = pq

    @pl.when(i != 0)
    def _acc():
        ps_ref[...] = ps_ref[...] + ps
        pq_ref[...] = pq_ref[...] + pq


def _edge_stats_body(x_ref, W_ref, ps_ref, pq_ref):
    y = jnp.dot(x_ref[...].astype(_BF16), W_ref[...],
                preferred_element_type=_F32)
    _stats_accum(pl.program_id(0), y, ps_ref, pq_ref)


def _edge_tmm_body(x_ref, p_ref, W_ref, t_ref, cs_ref, G_ref):
    i = pl.program_id(0)
    xv = x_ref[...]
    if xv.dtype != _BF16:
        xv = xv.astype(_BF16)
    x = jnp.dot(xv, W_ref[...], preferred_element_type=_F32) + p_ref[2]
    t16 = jnp.tanh(x * p_ref[0] + p_ref[1]).astype(_BF16)
    t_ref[...] = t16
    tf = t16.astype(_F32)
    yr = tf.reshape(-1, 8, 128)
    cs = jnp.sum(yr, axis=0)
    G = lax.dot_general(t16, t16, (((0,), (0,)), ((), ())),
                        preferred_element_type=_F32)

    @pl.when(i == 0)
    def _init():
        cs_ref[...] = cs
        G_ref[...] = G

    @pl.when(i != 0)
    def _acc():
        cs_ref[...] = cs_ref[...] + cs
        G_ref[...] = G_ref[...] + G


def _edge_last_body(x_ref, p_ref, W_ref, t_ref):
    x = jnp.dot(x_ref[...], W_ref[...],
                preferred_element_type=_F32) + p_ref[2]
    t_ref[...] = _bfr(jnp.tanh(x * p_ref[0] + p_ref[1]))


def _finale_body(S_ref, C_ref, Wf0_ref, Wf1_ref, mbf_ref, uW0_ref, uWa_ref,
                 uWb_ref, Wpred_ref, P_ref, out_ref):
    S = S_ref[...]
    cnt = C_ref[:, 0:1]

    hi = functools.partial(jnp.dot, preferred_element_type=_F32,
                           precision=lax.Precision.HIGHEST)
    agg_a = hi(S, Wf0_ref[...]) + cnt * mbf_ref[0]
    agg_b = (hi(S, Wf1_ref[...]) + cnt * mbf_ref[1]) / jnp.maximum(cnt, 1.0)
    x = agg_a + agg_b

    def bn(v, gm, bt):
        m = jnp.mean(v, axis=0)
        q = jnp.mean(v * v, axis=0)
        s = gm * lax.rsqrt(q - m * m + _EPS)
        return (v - m) * s + bt

    def mm(a, W_ref):
        return jnp.dot(a.astype(_BF16), W_ref[...],
                       preferred_element_type=_F32)

    x = mm(x, uW0_ref) + P_ref[0]
    x = mm(jnp.tanh(bn(x, P_ref[3], P_ref[4])), uWa_ref) + P_ref[1]
    x = mm(jnp.tanh(bn(x, P_ref[3], P_ref[4])), uWa_ref) + P_ref[1]
    x = mm(jnp.tanh(bn(x, P_ref[5], P_ref[6])), uWb_ref) + P_ref[2]
    x = jnp.tanh(bn(x, P_ref[7], P_ref[8]))
    out_ref[...] = mm(x, Wpred_ref) + P_ref[9]


def _edge_stats_pass(x, W16, block_e):
    E = x.shape[0]
    return pl.pallas_call(
        _edge_stats_body,
        grid=(E // block_e,),
        in_specs=[
            pl.BlockSpec((block_e, 128), lambda i: (i, 0)),
            pl.BlockSpec((128, 128), lambda i: (0, 0)),
        ],
        out_specs=[
            pl.BlockSpec((8, 128), lambda i: (0, 0)),
            pl.BlockSpec((8, 128), lambda i: (0, 0)),
        ],
        out_shape=[
            jax.ShapeDtypeStruct((8, 128), _F32),
            jax.ShapeDtypeStruct((8, 128), _F32),
        ],
        compiler_params=pltpu.CompilerParams(
            dimension_semantics=("arbitrary",)),
    )(x, W16)


def _edge_tmm_pass(x, params, W16, block_e):
    E = x.shape[0]
    return pl.pallas_call(
        _edge_tmm_body,
        grid=(E // block_e,),
        in_specs=[
            pl.BlockSpec((block_e, 128), lambda i: (i, 0)),
            pl.BlockSpec((8, 128), lambda i: (0, 0)),
            pl.BlockSpec((128, 128), lambda i: (0, 0)),
        ],
        out_specs=[
            pl.BlockSpec((block_e, 128), lambda i: (i, 0)),
            pl.BlockSpec((8, 128), lambda i: (0, 0)),
            pl.BlockSpec((128, 128), lambda i: (0, 0)),
        ],
        out_shape=[
            jax.ShapeDtypeStruct((E, 128), _BF16),
            jax.ShapeDtypeStruct((8, 128), _F32),
            jax.ShapeDtypeStruct((128, 128), _F32),
        ],
        compiler_params=pltpu.CompilerParams(
            dimension_semantics=("arbitrary",)),
    )(x, params, W16)


def _edge_last_pass(x, params, W16, block_e):
    E = x.shape[0]
    return pl.pallas_call(
        _edge_last_body,
        grid=(E // block_e,),
        in_specs=[
            pl.BlockSpec((block_e, 128), lambda i: (i, 0)),
            pl.BlockSpec((8, 128), lambda i: (0, 0)),
            pl.BlockSpec((128, 128), lambda i: (0, 0)),
        ],
        out_specs=pl.BlockSpec((block_e, 128), lambda i: (i, 0)),
        out_shape=jax.ShapeDtypeStruct((E, 128), _F32),
        compiler_params=pltpu.CompilerParams(
            dimension_semantics=("arbitrary",)),
    )(x, params, W16)


def _sc_gather_diff(h, src2d, dst2d):
    NCHT, CH = src2d.shape
    E = NCHT * CH
    EW = E // _NW
    NCH = EW // CH
    mesh = plsc.VectorSubcoreMesh(core_axis_name="c", subcore_axis_name="s")

    @functools.partial(
        pl.kernel,
        out_type=jax.ShapeDtypeStruct((E, 128), _F32),
        mesh=mesh,
        scratch_types=[
            pltpu.VMEM((NCH, CH), jnp.int32),
            pltpu.VMEM((NCH, CH), jnp.int32),
            pltpu.VMEM((CH, 128), _F32),
            pltpu.VMEM((CH, 128), _F32),
            pltpu.VMEM((CH, 128), _F32),
            pltpu.VMEM((CH, 128), _F32),
            pltpu.VMEM((CH, 128), _F32),
            pltpu.VMEM((CH, 128), _F32),
            pltpu.VMEM((CH, 128), _F32),
            pltpu.VMEM((CH, 128), _F32),
            pltpu.VMEM((CH, 128), _F32),
            pltpu.SemaphoreType.DMA,
            pltpu.SemaphoreType.DMA,
            pltpu.SemaphoreType.DMA,
            pltpu.SemaphoreType.DMA,
            pltpu.SemaphoreType.DMA,
            pltpu.SemaphoreType.DMA,
        ],
        compiler_params=pltpu.CompilerParams(use_tc_tiling_on_sc=False),
    )
    def k(h_hbm, src_hbm, dst_hbm, d_hbm, isrc, idst, gs0, gs1, gs2,
          gd0, gd1, gd2, db0, db1, db2, semg0, semg1, semg2,
          semw0, semw1, semw2):
        gs = (gs0, gs1, gs2)
        gd = (gd0, gd1, gd2)
        db = (db0, db1, db2)
        semg = (semg0, semg1, semg2)
        semw = (semw0, semw1, semw2)
        wid = lax.axis_index("s") * _NC + lax.axis_index("c")
        base = wid * EW
        row0 = wid * NCH

        pltpu.sync_copy(src_hbm.at[pl.ds(row0, NCH)], isrc)
        pltpu.sync_copy(dst_hbm.at[pl.ds(row0, NCH)], idst)

        def start_gathers(ci, s):
            pltpu.async_copy(h_hbm.at[isrc.at[ci]], gs[s], semg[s])
            pltpu.async_copy(h_hbm.at[idst.at[ci]], gd[s], semg[s])

        def wait_gathers(s):
            pltpu.make_async_copy(h_hbm.at[isrc.at[0]], gs[s], semg[s]).wait()
            pltpu.make_async_copy(h_hbm.at[idst.at[0]], gd[s], semg[s]).wait()

        def wait_write(s, ci):
            off = pl.multiple_of(base + ci * CH, 8)
            pltpu.make_async_copy(db[s], d_hbm.at[pl.ds(off, CH)],
                                  semw[s]).wait()

        start_gathers(0, 0)
        start_gathers(1, 1)

        def tri_body(t, _):
            for b in range(3):
                ci = 3 * t + b
                p = (b + 2) % 3

                @pl.when(ci < NCH)
                def _work():
                    wait_gathers(b)

                    @pl.when(ci >= 3)
                    def _ww():
                        wait_write(b, ci - 3)

                    def row_body(r, _):
                        for grp in range(8):
                            sl = pl.ds(grp * 16, 16)
                            db[b][r, sl] = gd[b][r, sl] - gs[b][r, sl]
                        return 0

                    lax.fori_loop(0, CH, row_body, 0)
                    off = pl.multiple_of(base + ci * CH, 8)
                    pltpu.async_copy(db[b], d_hbm.at[pl.ds(off, CH)], semw[b])

                    @pl.when(ci + 2 < NCH)
                    def _pf():
                        start_gathers(ci + 2, p)

            return 0

        lax.fori_loop(0, (NCH + 2) // 3, tri_body, 0)
        for last in range(max(NCH - 3, 0), NCH):
            wait_write(last % 3, last)

    return k(h, src2d, dst2d)


def _sc_scatter_add(t3, dst2d, n_nodes):
    NCHT, CH = dst2d.shape
    E = NCHT * CH
    EW = E // _NS
    NCH = EW // CH
    H = n_nodes // _NC
    HP = H + 128
    RPT = H // _NS // 8 * 8
    RLAST = H - (_NS - 1) * RPT
    CPT = RPT // 8
    CLAST = RLAST // 8
    mesh = plsc.VectorSubcoreMesh(core_axis_name="c", subcore_axis_name="s")

    @functools.partial(
        pl.kernel,
        out_type=[
            jax.ShapeDtypeStruct((n_nodes, 128), _F32),
            jax.ShapeDtypeStruct((n_nodes // 8, 128), _F32),
        ],
        mesh=mesh,
        scratch_types=[
            pltpu.VMEM((NCH, CH), jnp.int32),
            pltpu.VMEM((CH, 128), _F32),
            pltpu.VMEM((CH, 128), _F32),
            pltpu.VMEM((CH, 128), _F32),
            pltpu.VMEM((CH, 16), _F32),
            pltpu.VMEM((RLAST, 16), _F32),
            pltpu.VMEM((CLAST, 128), _F32),
            pltpu.VMEM_SHARED((HP, 128), _F32),
            pltpu.VMEM_SHARED((HP, 16), _F32),
            pltpu.SemaphoreType.DMA,
            pltpu.SemaphoreType.DMA,
            pltpu.SemaphoreType.DMA,
            pltpu.SemaphoreType.DMA,
            pltpu.SemaphoreType.DMA,
            pltpu.SemaphoreType.DMA,
            pltpu.SemaphoreType.DMA,
            pltpu.SemaphoreType.DMA,
            pltpu.SemaphoreType.DMA,
        ],
        compiler_params=pltpu.CompilerParams(use_tc_tiling_on_sc=False),
    )
    def k(t3_hbm, dst_hbm, zS_hbm, zC_hbm, outS, outC,
          idxa, tbuf0, tbuf1, tbuf2, onesv, bC, bC2, S_sh, C_sh,
          semt0, semt1, semt2, sems0, sems1, sems2, semo0, semo1, semo2):
        tbuf = (tbuf0, tbuf1, tbuf2)
        semt = (semt0, semt1, semt2)
        sems = (sems0, sems1, sems2)
        semo = (semo0, semo1, semo2)
        cid = lax.axis_index("c")
        sid = lax.axis_index("s")
        node0 = cid * H
        ro = sid * RPT
        row0 = sid * NCH

        one = jnp.full((16,), 1.0, _F32)

        def ones_body(r, _):
            onesv[r, :] = one
            return 0

        lax.fori_loop(0, CH, ones_body, 0)

        @pl.when(sid == 0)
        def _z0():
            pltpu.sync_copy(zS_hbm, S_sh)
            pltpu.sync_copy(zC_hbm, C_sh)

        pltpu.sync_copy(dst_hbm.at[pl.ds(row0, NCH)], idxa)

        def adj_body(r, _):
            for j in range(CH // 16):
                sl = pl.ds(j * 16, 16)
                local = idxa[r, sl] - node0
                ok = (local >= 0) & (local < H)
                idxa[r, sl] = jnp.where(ok, local, H)
            return 0

        lax.fori_loop(0, NCH, adj_body, 0)
        plsc.subcore_barrier()

        def load_t3(ci, s):
            off = pl.multiple_of(sid * EW + ci * CH, 8)
            pltpu.async_copy(t3_hbm.at[pl.ds(off, CH)], tbuf[s], semt[s])

        def wait_load(s):
            pltpu.make_async_copy(t3_hbm.at[pl.ds(0, CH)], tbuf[s],
                                  semt[s]).wait()

        def wait_scats(s):
            pltpu.make_async_copy(tbuf[s], S_sh.at[pl.ds(0, CH)],
                                  sems[s]).wait()
            pltpu.make_async_copy(onesv, C_sh.at[pl.ds(0, CH)],
                                  semo[s]).wait()

        load_t3(0, 0)
        load_t3(1, 1)

        def tri_body(t, _):
            for b in range(3):
                ci = 3 * t + b
                p = (b + 2) % 3

                @pl.when(ci < NCH)
                def _work():
                    wait_load(b)
                    pltpu.async_copy(tbuf[b], S_sh.at[idxa.at[ci]], sems[b],
                                     add=True)
                    pltpu.async_copy(onesv, C_sh.at[idxa.at[ci]], semo[b],
                                     add=True)

                    @pl.when(ci + 2 < NCH)
                    def _pf():
                        @pl.when(ci >= 1)
                        def _ws():
                            wait_scats(p)

                        load_t3(ci + 2, p)

            return 0

        lax.fori_loop(0, (NCH + 2) // 3, tri_body, 0)
        for last in range(max(NCH - 3, 0), NCH):
            wait_scats(last % 3)
        plsc.subcore_barrier()

        @pl.when(sid != _NS - 1)
        def _r0():
            pltpu.sync_copy(S_sh.at[pl.ds(ro, RPT)],
                            outS.at[pl.ds(node0 + ro, RPT)])
            pltpu.sync_copy(C_sh.at[pl.ds(ro, RPT)], bC.at[pl.ds(0, RPT)])

        @pl.when(sid == _NS - 1)
        def _r1():
            pltpu.sync_copy(S_sh.at[pl.ds(ro, RLAST)],
                            outS.at[pl.ds(node0 + ro, RLAST)])
            pltpu.sync_copy(C_sh.at[pl.ds(ro, RLAST)], bC)

        def repack_body(r, _):
            bC2[r // 8, pl.ds((r % 8) * 16, 16)] = bC[r, :]
            return 0

        lax.fori_loop(0, RLAST, repack_body, 0)
        co = cid * (H // 8) + sid * CPT

        @pl.when(sid != _NS - 1)
        def _c0():
            pltpu.sync_copy(bC2.at[pl.ds(0, CPT)], outC.at[pl.ds(co, CPT)])

        @pl.when(sid == _NS - 1)
        def _c1():
            pltpu.sync_copy(bC2, outC.at[pl.ds(co, CLAST)])

    zS = jnp.zeros((HP, 128), _F32)
    zC = jnp.zeros((HP, 16), _F32)
    return k(t3, dst2d, zS, zC)


def _affine_from_stats(ps, pq, count, gamma, beta):
    m = jnp.sum(ps, 0) / count
    var = jnp.sum(pq, 0) / count - m * m
    scale = gamma * lax.rsqrt(var + _EPS)
    return scale, beta - m * scale


def _affine_from_gram(cs, G, Wr, badd, count, gamma, beta):
    hi = functools.partial(jnp.dot, precision=lax.Precision.HIGHEST)
    mu = hi(jnp.sum(cs, 0) / count, Wr)
    T = hi(G / count, Wr)
    ex2 = jnp.sum(Wr * T, axis=0)
    var = ex2 - mu * mu
    scale = gamma * lax.rsqrt(var + _EPS)
    return scale, beta - (mu + badd) * scale


def kernel(pos, vel, edge_index, W_in, b_in, mW0, mb0, mWa, mba, mWf, mbf,
           mg_a, mbe_a, mg_b, mbe_b, uW0, ub0, uWa, uba, uWb, ubb,
           ug_a, ube_a, ug_b, ube_b, ug_c, ube_c, W_pred, b_pred):
    N = pos.shape[0]
    E = edge_index.shape[1]
    src = edge_index[0]
    dst = edge_index[1]
    BE = 16000
    fE = jnp.float32(E)

    pv = jnp.zeros((N, 128), _F32)
    pv = pv.at[:, 0:2].set(pos).at[:, 2:4].set(vel)
    Wp = jnp.zeros((128, 128), _F32).at[0:4, :].set(W_in)
    h = pl.pallas_call(
        _node_prologue_body,
        out_shape=jax.ShapeDtypeStruct((N, 128), _F32),
    )(pv, Wp, b_in.reshape(1, 128))

    src2d = src.reshape(E // _CH, _CH)
    dst2d = dst.reshape(E // _CH, _CH)
    d = _sc_gather_diff(h, src2d, dst2d)

    def pack_p(scale, bias, bout):
        p = jnp.zeros((8, 128), _F32)
        return p.at[0].set(scale).at[1].set(bias).at[2].set(bout)

    mW0_16 = mW0.astype(_BF16)
    mWa_16 = mWa.astype(_BF16)
    zeros128 = jnp.zeros((128,), _F32)

    ps1, pq1 = _edge_stats_pass(d, mW0_16, BE)
    s1, b1 = _affine_from_stats(ps1, pq1, fE, mg_a, mbe_a)

    t1b, cs2, G2 = _edge_tmm_pass(d, pack_p(s1, b1, zeros128), mW0_16, BE)
    s2, b2 = _affine_from_gram(cs2, G2, _bfr(mWa), mba, fE, mg_a, mbe_a)

    t2b, cs3, G3 = _edge_tmm_pass(t1b, pack_p(s2, b2, mba), mWa_16, BE)
    s3, b3 = _affine_from_gram(cs3, G3, _bfr(mWa), mba, fE, mg_b, mbe_b)

    t3 = _edge_last_pass(t2b, pack_p(s3, b3, mba), mWa_16, BE)

    S_seg, C_raw = _sc_scatter_add(t3, dst2d, N)
    cnt = jnp.broadcast_to(C_raw.reshape(N, 16)[:, 0:1], (N, 8))

    P = jnp.zeros((16, 128), _F32)
    for i, v in enumerate([ub0, uba, ubb, ug_a, ube_a, ug_b, ube_b,
                           ug_c, ube_c]):
        P = P.at[i].set(v)
    P = P.at[9, 0:2].set(b_pred)
    Wpred16 = jnp.zeros((128, 128), _F32).at[:, 0:2].set(W_pred).astype(_BF16)
    out = pl.pallas_call(
        _finale_body,
        out_shape=jax.ShapeDtypeStruct((N, 128), _F32),
    )(S_seg, cnt, _bfr(mWf[:, :128]), _bfr(mWf[:, 128:]),
      mbf.reshape(2, 128), uW0.astype(_BF16), uWa.astype(_BF16),
      uWb.astype(_BF16), Wpred16, P)
    return out[:, 0:2]


# speedup vs baseline: 1.7397x; 1.0095x over previous
import functools

import jax
import jax.numpy as jnp
from jax import lax
from jax.experimental import pallas as pl
from jax.experimental.pallas import tpu as pltpu
from jax.experimental.pallas import tpu_sc as plsc

_EPS = 1e-5
_NC = 2
_NS = 16
_NW = _NC * _NS
_CH = 80

_F32 = jnp.float32
_BF16 = jnp.bfloat16


def _bfr(x):
    return x.astype(_BF16).astype(_F32)


def _node_prologue_body(pv_ref, Wp_ref, bp_ref, h_ref):
    h = jnp.dot(pv_ref[...].astype(_BF16), Wp_ref[...].astype(_BF16),
                preferred_element_type=_F32)
    h_ref[...] = h + bp_ref[0]


def _stats_accum(i, y, ps_ref, pq_ref):
    yr = y.reshape(-1, 8, 128)
    ps = jnp.sum(yr, axis=0)
    pq = jnp.sum(yr * yr, axis=0)

    @pl.when(i == 0)
    def _init():
        ps_ref[...] = ps
        pq_ref[...] = pq

    @pl.when(i != 0)
    def _acc():
        ps_ref[...] = ps_ref[...] + ps
        pq_ref[...] = pq_ref[...] + pq


def _edge_stats_body(x_ref, W_ref, ps_ref, pq_ref):
    y = jnp.dot(x_ref[...].astype(_BF16), W_ref[...],
                preferred_element_type=_F32)
    _stats_accum(pl.program_id(0), y, ps_ref, pq_ref)


def _edge_tmm_body(x_ref, p_ref, W_ref, t_ref, cs_ref, G_ref):
    i = pl.program_id(0)
    xv = x_ref[...]
    if xv.dtype != _BF16:
        xv = xv.astype(_BF16)
    x = jnp.dot(xv, W_ref[...], preferred_element_type=_F32) + p_ref[2]
    t16 = jnp.tanh(x * p_ref[0] + p_ref[1]).astype(_BF16)
    t_ref[...] = t16
    tf = t16.astype(_F32)
    yr = tf.reshape(-1, 8, 128)
    cs = jnp.sum(yr, axis=0)
    G = lax.dot_general(t16, t16, (((0,), (0,)), ((), ())),
                        preferred_element_type=_F32)

    @pl.when(i == 0)
    def _init():
        cs_ref[...] = cs
        G_ref[...] = G

    @pl.when(i != 0)
    def _acc():
        cs_ref[...] = cs_ref[...] + cs
        G_ref[...] = G_ref[...] + G


def _edge_last_body(x_ref, p_ref, W_ref, t_ref):
    x = jnp.dot(x_ref[...], W_ref[...],
                preferred_element_type=_F32) + p_ref[2]
    t_ref[...] = _bfr(jnp.tanh(x * p_ref[0] + p_ref[1]))


def _finale_body(S_ref, C_ref, Wf0_ref, Wf1_ref, mbf_ref, uW0_ref, uWa_ref,
                 uWb_ref, Wpred_ref, P_ref, out_ref):
    S = S_ref[...]
    cnt = C_ref[:, 0:1]

    hi = functools.partial(jnp.dot, preferred_element_type=_F32,
                           precision=lax.Precision.HIGHEST)
    agg_a = hi(S, Wf0_ref[...]) + cnt * mbf_ref[0]
    agg_b = (hi(S, Wf1_ref[...]) + cnt * mbf_ref[1]) / jnp.maximum(cnt, 1.0)
    x = agg_a + agg_b

    def bn(v, gm, bt):
        m = jnp.mean(v, axis=0)
        q = jnp.mean(v * v, axis=0)
        s = gm * lax.rsqrt(q - m * m + _EPS)
        return (v - m) * s + bt

    def mm(a, W_ref):
        return jnp.dot(a.astype(_BF16), W_ref[...],
                       preferred_element_type=_F32)

    x = mm(x, uW0_ref) + P_ref[0]
    x = mm(jnp.tanh(bn(x, P_ref[3], P_ref[4])), uWa_ref) + P_ref[1]
    x = mm(jnp.tanh(bn(x, P_ref[3], P_ref[4])), uWa_ref) + P_ref[1]
    x = mm(jnp.tanh(bn(x, P_ref[5], P_ref[6])), uWb_ref) + P_ref[2]
    x = jnp.tanh(bn(x, P_ref[7], P_ref[8]))
    out_ref[...] = mm(x, Wpred_ref) + P_ref[9]


def _edge_stats_pass(x, W16, block_e):
    E = x.shape[0]
    return pl.pallas_call(
        _edge_stats_body,
        grid=(E // block_e,),
        in_specs=[
            pl.BlockSpec((block_e, 128), lambda i: (i, 0)),
            pl.BlockSpec((128, 128), lambda i: (0, 0)),
        ],
        out_specs=[
            pl.BlockSpec((8, 128), lambda i: (0, 0)),
            pl.BlockSpec((8, 128), lambda i: (0, 0)),
        ],
        out_shape=[
            jax.ShapeDtypeStruct((8, 128), _F32),
            jax.ShapeDtypeStruct((8, 128), _F32),
        ],
        compiler_params=pltpu.CompilerParams(
            dimension_semantics=("arbitrary",)),
    )(x, W16)


def _edge_tmm_pass(x, params, W16, block_e):
    E = x.shape[0]
    return pl.pallas_call(
        _edge_tmm_body,
        grid=(E // block_e,),
        in_specs=[
            pl.BlockSpec((block_e, 128), lambda i: (i, 0)),
            pl.BlockSpec((8, 128), lambda i: (0, 0)),
            pl.BlockSpec((128, 128), lambda i: (0, 0)),
        ],
        out_specs=[
            pl.BlockSpec((block_e, 128), lambda i: (i, 0)),
            pl.BlockSpec((8, 128), lambda i: (0, 0)),
            pl.BlockSpec((128, 128), lambda i: (0, 0)),
        ],
        out_shape=[
            jax.ShapeDtypeStruct((E, 128), _BF16),
            jax.ShapeDtypeStruct((8, 128), _F32),
            jax.ShapeDtypeStruct((128, 128), _F32),
        ],
        compiler_params=pltpu.CompilerParams(
            dimension_semantics=("arbitrary",)),
    )(x, params, W16)


def _edge_last_pass(x, params, W16, block_e):
    E = x.shape[0]
    return pl.pallas_call(
        _edge_last_body,
        grid=(E // block_e,),
        in_specs=[
            pl.BlockSpec((block_e, 128), lambda i: (i, 0)),
            pl.BlockSpec((8, 128), lambda i: (0, 0)),
            pl.BlockSpec((128, 128), lambda i: (0, 0)),
        ],
        out_specs=pl.BlockSpec((block_e, 128), lambda i: (i, 0)),
        out_shape=jax.ShapeDtypeStruct((E, 128), _F32),
        compiler_params=pltpu.CompilerParams(
            dimension_semantics=("arbitrary",)),
    )(x, params, W16)


def _sc_gather_diff(h, src2d, dst2d):
    NCHT, CH = src2d.shape
    E = NCHT * CH
    EW = E // _NW
    NCH = EW // CH
    mesh = plsc.VectorSubcoreMesh(core_axis_name="c", subcore_axis_name="s")

    @functools.partial(
        pl.kernel,
        out_type=jax.ShapeDtypeStruct((E, 128), _F32),
        mesh=mesh,
        scratch_types=[
            pltpu.VMEM((NCH, CH), jnp.int32),
            pltpu.VMEM((NCH, CH), jnp.int32),
            pltpu.VMEM((CH, 128), _F32),
            pltpu.VMEM((CH, 128), _F32),
            pltpu.VMEM((CH, 128), _F32),
            pltpu.VMEM((CH, 128), _F32),
            pltpu.VMEM((CH, 128), _F32),
            pltpu.VMEM((CH, 128), _F32),
            pltpu.VMEM((CH, 128), _F32),
            pltpu.VMEM((CH, 128), _F32),
            pltpu.VMEM((CH, 128), _F32),
            pltpu.SemaphoreType.DMA,
            pltpu.SemaphoreType.DMA,
            pltpu.SemaphoreType.DMA,
            pltpu.SemaphoreType.DMA,
            pltpu.SemaphoreType.DMA,
            pltpu.SemaphoreType.DMA,
        ],
        compiler_params=pltpu.CompilerParams(use_tc_tiling_on_sc=False),
    )
    def k(h_hbm, src_hbm, dst_hbm, d_hbm, isrc, idst, gs0, gs1, gs2,
          gd0, gd1, gd2, db0, db1, db2, semg0, semg1, semg2,
          semw0, semw1, semw2):
        gs = (gs0, gs1, gs2)
        gd = (gd0, gd1, gd2)
        db = (db0, db1, db2)
        semg = (semg0, semg1, semg2)
        semw = (semw0, semw1, semw2)
        wid = lax.axis_index("s") * _NC + lax.axis_index("c")
        base = wid * EW
        row0 = wid * NCH

        pltpu.sync_copy(src_hbm.at[pl.ds(row0, NCH)], isrc)
        pltpu.sync_copy(dst_hbm.at[pl.ds(row0, NCH)], idst)

        def start_gathers(ci, s):
            pltpu.async_copy(h_hbm.at[isrc.at[ci]], gs[s], semg[s])
            pltpu.async_copy(h_hbm.at[idst.at[ci]], gd[s], semg[s])

        def wait_gathers(s):
            pltpu.make_async_copy(h_hbm.at[isrc.at[0]], gs[s], semg[s]).wait()
            pltpu.make_async_copy(h_hbm.at[idst.at[0]], gd[s], semg[s]).wait()

        def wait_write(s, ci):
            off = pl.multiple_of(base + ci * CH, 8)
            pltpu.make_async_copy(db[s], d_hbm.at[pl.ds(off, CH)],
                                  semw[s]).wait()

        start_gathers(0, 0)
        start_gathers(1, 1)

        def tri_body(t, _):
            for b in range(3):
                ci = 3 * t + b
                p = (b + 2) % 3

                @pl.when(ci < NCH)
                def _work():
                    wait_gathers(b)

                    @pl.when(ci >= 3)
                    def _ww():
                        wait_write(b, ci - 3)

                    def row_body(r, _):
                        for grp in range(8):
                            sl = pl.ds(grp * 16, 16)
                            db[b][r, sl] = gd[b][r, sl] - gs[b][r, sl]
                        return 0

                    lax.fori_loop(0, CH, row_body, 0)
                    off = pl.multiple_of(base + ci * CH, 8)
                    pltpu.async_copy(db[b], d_hbm.at[pl.ds(off, CH)], semw[b])

                    @pl.when(ci + 2 < NCH)
                    def _pf():
                        start_gathers(ci + 2, p)

            return 0

        lax.fori_loop(0, (NCH + 2) // 3, tri_body, 0)
        for last in range(max(NCH - 3, 0), NCH):
            wait_write(last % 3, last)

    return k(h, src2d, dst2d)


def _sc_scatter_add(t3, dst2d, n_nodes):
    NCHT, CH = dst2d.shape
    E = NCHT * CH
    EW = E // _NS
    NCH = EW // CH
    H = n_nodes // _NC
    HP = H + 128
    RPT = H // _NS // 8 * 8
    RLAST = H - (_NS - 1) * RPT
    CPT = RPT // 8
    CLAST = RLAST // 8
    mesh = plsc.VectorSubcoreMesh(core_axis_name="c", subcore_axis_name="s")

    @functools.partial(
        pl.kernel,
        out_type=[
            jax.ShapeDtypeStruct((n_nodes, 128), _F32),
            jax.ShapeDtypeStruct((n_nodes // 8, 128), _F32),
        ],
        mesh=mesh,
        scratch_types=[
            pltpu.VMEM((NCH, CH), jnp.int32),
            pltpu.VMEM((CH, 128), _F32),
            pltpu.VMEM((CH, 128), _F32),
            pltpu.VMEM((CH, 128), _F32),
            pltpu.VMEM((CH, 16), _F32),
            pltpu.VMEM((RLAST, 16), _F32),
            pltpu.VMEM((CLAST, 128), _F32),
            pltpu.VMEM_SHARED((HP, 128), _F32),
            pltpu.VMEM_SHARED((HP, 16), _F32),
            pltpu.SemaphoreType.DMA,
            pltpu.SemaphoreType.DMA,
            pltpu.SemaphoreType.DMA,
            pltpu.SemaphoreType.DMA,
            pltpu.SemaphoreType.DMA,
            pltpu.SemaphoreType.DMA,
            pltpu.SemaphoreType.DMA,
            pltpu.SemaphoreType.DMA,
            pltpu.SemaphoreType.DMA,
        ],
        compiler_params=pltpu.CompilerParams(use_tc_tiling_on_sc=False),
    )
    def k(t3_hbm, dst_hbm, zS_hbm, zC_hbm, outS, outC,
          idxa, tbuf0, tbuf1, tbuf2, onesv, bC, bC2, S_sh, C_sh,
          semt0, semt1, semt2, sems0, sems1, sems2, semo0, semo1, semo2):
        tbuf = (tbuf0, tbuf1, tbuf2)
        semt = (semt0, semt1, semt2)
        sems = (sems0, sems1, sems2)
        semo = (semo0, semo1, semo2)
        cid = lax.axis_index("c")
        sid = lax.axis_index("s")
        node0 = cid * H
        ro = sid * RPT
        row0 = sid * NCH

        one = jnp.full((16,), 1.0, _F32)

        def ones_body(r, _):
            onesv[r, :] = one
            return 0

        lax.fori_loop(0, CH, ones_body, 0)

        @pl.when(sid == 0)
        def _z0():
            pltpu.sync_copy(zS_hbm, S_sh)
            pltpu.sync_copy(zC_hbm, C_sh)

        pltpu.sync_copy(dst_hbm.at[pl.ds(row0, NCH)], idxa)

        def adj_body(r, _):
            for j in range(CH // 16):
                sl = pl.ds(j * 16, 16)
                local = idxa[r, sl] - node0
                ok = (local >= 0) & (local < H)
                idxa[r, sl] = jnp.where(ok, local, H)
            return 0

        lax.fori_loop(0, NCH, adj_body, 0)
        plsc.subcore_barrier()

        def load_t3(ci, s):
            off = pl.multiple_of(sid * EW + ci * CH, 8)
            pltpu.async_copy(t3_hbm.at[pl.ds(off, CH)], tbuf[s], semt[s])

        def wait_load(s):
            pltpu.make_async_copy(t3_hbm.at[pl.ds(0, CH)], tbuf[s],
                                  semt[s]).wait()

        def wait_scats(s):
            pltpu.make_async_copy(tbuf[s], S_sh.at[pl.ds(0, CH)],
                                  sems[s]).wait()
            pltpu.make_async_copy(onesv, C_sh.at[pl.ds(0, CH)],
                                  semo[s]).wait()

        load_t3(0, 0)
        load_t3(1, 1)

        def tri_body(t, _):
            for b in range(3):
                ci = 3 * t + b
                p = (b + 2) % 3

                @pl.when(ci < NCH)
                def _work():
                    wait_load(b)
                    pltpu.async_copy(tbuf[b], S_sh.at[idxa.at[ci]], sems[b],
                                     add=True)
                    pltpu.async_copy(onesv, C_sh.at[idxa.at[ci]], semo[b],
                                     add=True)

                    @pl.when(ci + 2 < NCH)
                    def _pf():
                        @pl.when(ci >= 1)
                        def _ws():
                            wait_scats(p)

                        load_t3(ci + 2, p)

            return 0

        lax.fori_loop(0, (NCH + 2) // 3, tri_body, 0)
        for last in range(max(NCH - 3, 0), NCH):
            wait_scats(last % 3)
        plsc.subcore_barrier()

        @pl.when(sid != _NS - 1)
        def _r0():
            pltpu.sync_copy(S_sh.at[pl.ds(ro, RPT)],
                            outS.at[pl.ds(node0 + ro, RPT)])
            pltpu.sync_copy(C_sh.at[pl.ds(ro, RPT)], bC.at[pl.ds(0, RPT)])

        @pl.when(sid == _NS - 1)
        def _r1():
            pltpu.sync_copy(S_sh.at[pl.ds(ro, RLAST)],
                            outS.at[pl.ds(node0 + ro, RLAST)])
            pltpu.sync_copy(C_sh.at[pl.ds(ro, RLAST)], bC)

        def repack_body(r, _):
            bC2[r // 8, pl.ds((r % 8) * 16, 16)] = bC[r, :]
            return 0

        lax.fori_loop(0, RLAST, repack_body, 0)
        co = cid * (H // 8) + sid * CPT

        @pl.when(sid != _NS - 1)
        def _c0():
            pltpu.sync_copy(bC2.at[pl.ds(0, CPT)], outC.at[pl.ds(co, CPT)])

        @pl.when(sid == _NS - 1)
        def _c1():
            pltpu.sync_copy(bC2, outC.at[pl.ds(co, CLAST)])

    zS = jnp.zeros((HP, 128), _F32)
    zC = jnp.zeros((HP, 16), _F32)
    return k(t3, dst2d, zS, zC)


def _affine_from_stats(ps, pq, count, gamma, beta):
    m = jnp.sum(ps, 0) / count
    var = jnp.sum(pq, 0) / count - m * m
    scale = gamma * lax.rsqrt(var + _EPS)
    return scale, beta - m * scale


def _affine_from_gram(cs, G, Wr, badd, count, gamma, beta):
    hi = functools.partial(jnp.dot, precision=lax.Precision.HIGHEST)
    mu = hi(jnp.sum(cs, 0) / count, Wr)
    T = hi(G / count, Wr)
    ex2 = jnp.sum(Wr * T, axis=0)
    var = ex2 - mu * mu
    scale = gamma * lax.rsqrt(var + _EPS)
    return scale, beta - (mu + badd) * scale


def kernel(pos, vel, edge_index, W_in, b_in, mW0, mb0, mWa, mba, mWf, mbf,
           mg_a, mbe_a, mg_b, mbe_b, uW0, ub0, uWa, uba, uWb, ubb,
           ug_a, ube_a, ug_b, ube_b, ug_c, ube_c, W_pred, b_pred):
    N = pos.shape[0]
    E = edge_index.shape[1]
    src = edge_index[0]
    dst = edge_index[1]
    BE = 20000
    fE = jnp.float32(E)

    pv = jnp.zeros((N, 128), _F32)
    pv = pv.at[:, 0:2].set(pos).at[:, 2:4].set(vel)
    Wp = jnp.zeros((128, 128), _F32).at[0:4, :].set(W_in)
    h = pl.pallas_call(
        _node_prologue_body,
        out_shape=jax.ShapeDtypeStruct((N, 128), _F32),
    )(pv, Wp, b_in.reshape(1, 128))

    src2d = src.reshape(E // _CH, _CH)
    dst2d = dst.reshape(E // _CH, _CH)
    d = _sc_gather_diff(h, src2d, dst2d)

    def pack_p(scale, bias, bout):
        p = jnp.zeros((8, 128), _F32)
        return p.at[0].set(scale).at[1].set(bias).at[2].set(bout)

    mW0_16 = mW0.astype(_BF16)
    mWa_16 = mWa.astype(_BF16)
    zeros128 = jnp.zeros((128,), _F32)

    ps1, pq1 = _edge_stats_pass(d, mW0_16, BE)
    s1, b1 = _affine_from_stats(ps1, pq1, fE, mg_a, mbe_a)

    t1b, cs2, G2 = _edge_tmm_pass(d, pack_p(s1, b1, zeros128), mW0_16, BE)
    s2, b2 = _affine_from_gram(cs2, G2, _bfr(mWa), mba, fE, mg_a, mbe_a)

    t2b, cs3, G3 = _edge_tmm_pass(t1b, pack_p(s2, b2, mba), mWa_16, BE)
    s3, b3 = _affine_from_gram(cs3, G3, _bfr(mWa), mba, fE, mg_b, mbe_b)

    t3 = _edge_last_pass(t2b, pack_p(s3, b3, mba), mWa_16, BE)

    S_seg, C_raw = _sc_scatter_add(t3, dst2d, N)
    cnt = jnp.broadcast_to(C_raw.reshape(N, 16)[:, 0:1], (N, 8))

    P = jnp.zeros((16, 128), _F32)
    for i, v in enumerate([ub0, uba, ubb, ug_a, ube_a, ug_b, ube_b,
                           ug_c, ube_c]):
        P = P.at[i].set(v)
    P = P.at[9, 0:2].set(b_pred)
    Wpred16 = jnp.zeros((128, 128), _F32).at[:, 0:2].set(W_pred).astype(_BF16)
    out = pl.pallas_call(
        _finale_body,
        out_shape=jax.ShapeDtypeStruct((N, 128), _F32),
    )(S_seg, cnt, _bfr(mWf[:, :128]), _bfr(mWf[:, 128:]),
      mbf.reshape(2, 128), uW0.astype(_BF16), uWa.astype(_BF16),
      uWb.astype(_BF16), Wpred16, P)
    return out[:, 0:2]
